# Initial kernel scaffold; baseline (speedup 1.0000x reference)
import jax
import jax.numpy as jnp
from jax.experimental import pallas as pl


def kernel(atom_rep_0, atom_rep_1, atom_rep_2, probe_edges_features, params, probe_edges):
    raise NotImplementedError("write your pallas kernel here")


# speedup vs baseline: 1.4285x; 1.4285x over previous
import functools
import math

import jax
import jax.numpy as jnp
from jax import lax
from jax.experimental import pallas as pl
from jax.experimental.pallas import tpu as pltpu
from jax.experimental.pallas import tpu_sc as plsc

H = 128
CUTOFF = 5.0
STEP = 0.1
ES = 50
ESP = 64
NI = 3
NA = 10000
NP = 8000
E = 128000
PB = 8192
NC = 2
NS = 16
NW = NC * NS
EP = 131072
EPW = EP // NW
CH = 128
NCH = EPW // CH
LOG2 = float(math.log(2.0))
BE = 2048
BP = 1024


def _ssp(x):
    return jnp.maximum(x, 0.0) + jnp.log1p(jnp.exp(-jnp.abs(x))) - LOG2


def _pre_body(a_ref, w_ref, o_ref):
    o_ref[0] = jnp.dot(a_ref[0], w_ref[0], preferred_element_type=jnp.float32)


def _precompute_A(atoms_s, waT_s):
    return pl.pallas_call(
        _pre_body,
        grid=(NI,),
        in_specs=[
            pl.BlockSpec((1, NA, H), lambda i: (i, 0, 0)),
            pl.BlockSpec((1, H, H), lambda i: (i, 0, 0)),
        ],
        out_specs=pl.BlockSpec((1, NA, H), lambda i: (i, 0, 0)),
        out_shape=jax.ShapeDtypeStruct((NI, NA, H), jnp.float32),
    )(atoms_s, waT_s)


def _edge_body(h_ref, d_ref, b1_ref, m2_ref, b2_ref, f1_ref, bf1_ref,
               f2_ref, bf2_ref, mu_ref, o_ref):
    d = d_ref[...]
    h = _ssp(h_ref[...] + b1_ref[...])
    m = jnp.dot(h, m2_ref[...], preferred_element_type=jnp.float32) + b2_ref[...]
    es = jnp.exp((-1.0 / (2.0 * STEP * STEP)) * (d - mu_ref[...]) ** 2)
    t = _ssp(jnp.dot(es, f1_ref[...], preferred_element_type=jnp.float32)
             + bf1_ref[...])
    fw = jnp.dot(t, f2_ref[...], preferred_element_type=jnp.float32) + bf2_ref[...]
    cc = jnp.where(d < CUTOFF, 0.5 * (jnp.cos((jnp.pi / CUTOFF) * d) + 1.0), 0.0)
    o_ref[...] = m * (fw * cc)


def _edge_mlp(hidden, d_pad, b1, m2T, b2, f1T, bf1, f2T, bf2, mu_pad):
    full = lambda r, c: pl.BlockSpec((r, c), lambda i: (0, 0))
    return pl.pallas_call(
        _edge_body,
        grid=(EP // BE,),
        in_specs=[
            pl.BlockSpec((BE, H), lambda i: (i, 0)),
            pl.BlockSpec((BE, 1), lambda i: (i, 0)),
            full(1, H), full(H, H), full(1, H),
            full(ESP, H), full(1, H), full(H, H), full(1, H), full(1, ESP),
        ],
        out_specs=pl.BlockSpec((BE, H), lambda i: (i, 0)),
        out_shape=jax.ShapeDtypeStruct((EP, H), jnp.float32),
    )(hidden, d_pad, b1, m2T, b2, f1T, bf1, f2T, bf2, mu_pad)


def _probe_common(p_ref, ps_ref, g1, bg1, g2, bg2, s1, bs1, s2, bs2):
    msgsum = p_ref[0] + p_ref[1]
    probe = ps_ref[...]
    gates = jax.nn.sigmoid(
        jnp.dot(_ssp(jnp.dot(probe, g1[...], preferred_element_type=jnp.float32)
                     + bg1[...]), g2[...], preferred_element_type=jnp.float32)
        + bg2[...])
    trans = (jnp.dot(_ssp(jnp.dot(msgsum, s1[...],
                                  preferred_element_type=jnp.float32) + bs1[...]),
                     s2[...], preferred_element_type=jnp.float32) + bs2[...])
    return probe * gates + (1.0 - gates) * trans


def _probe_step_body(p_ref, ps_ref, g1, bg1, g2, bg2, s1, bs1, s2, bs2, wbn,
                     np_ref, pn_ref):
    newp = _probe_common(p_ref, ps_ref, g1, bg1, g2, bg2, s1, bs1, s2, bs2)
    np_ref[...] = newp
    pn_ref[...] = jnp.dot(newp, wbn[...], preferred_element_type=jnp.float32)


def _probe_final_body(p_ref, ps_ref, g1, bg1, g2, bg2, s1, bs1, s2, bs2,
                      r1, br1, r2, br2, o_ref):
    newp = _probe_common(p_ref, ps_ref, g1, bg1, g2, bg2, s1, bs1, s2, bs2)
    ro = jnp.dot(_ssp(jnp.dot(newp, r1[...], preferred_element_type=jnp.float32)
                      + br1[...]), r2[...], preferred_element_type=jnp.float32)
    o_ref[...] = ro + br2[...]


def _probe_step(partials, probe, gw, sw, wbnT):
    full = lambda r, c: pl.BlockSpec((r, c), lambda i: (0, 0))
    row = pl.BlockSpec((BP, H), lambda i: (i, 0))
    return pl.pallas_call(
        _probe_step_body,
        grid=(PB // BP,),
        in_specs=[
            pl.BlockSpec((NC, BP, H), lambda i: (0, i, 0)), row,
            full(H, H), full(1, H), full(H, H), full(1, H),
            full(H, H), full(1, H), full(H, H), full(1, H),
            full(H, H),
        ],
        out_specs=[row, row],
        out_shape=[jax.ShapeDtypeStruct((PB, H), jnp.float32),
                   jax.ShapeDtypeStruct((PB, H), jnp.float32)],
    )(partials, probe, *gw, *sw, wbnT)


def _probe_final(partials, probe, gw, sw, rw):
    full = lambda r, c: pl.BlockSpec((r, c), lambda i: (0, 0))
    row = pl.BlockSpec((BP, H), lambda i: (i, 0))
    return pl.pallas_call(
        _probe_final_body,
        grid=(PB // BP,),
        in_specs=[
            pl.BlockSpec((NC, BP, H), lambda i: (0, i, 0)), row,
            full(H, H), full(1, H), full(H, H), full(1, H),
            full(H, H), full(1, H), full(H, H), full(1, H),
            full(H, H), full(1, H), full(H, 8), full(1, 8),
        ],
        out_specs=pl.BlockSpec((BP, 8), lambda i: (i, 0)),
        out_shape=jax.ShapeDtypeStruct((PB, 8), jnp.float32),
    )(partials, probe, *gw, *sw, *rw)


def _sc_mesh():
    return plsc.VectorSubcoreMesh(core_axis_name="c", subcore_axis_name="s",
                                  num_cores=NC)


def _sc_gather1(table, src3):
    @functools.partial(
        pl.kernel,
        out_type=jax.ShapeDtypeStruct((EP, H), jnp.float32),
        mesh=_sc_mesh(),
        scratch_types=[
            pltpu.VMEM((NCH, CH), jnp.int32),
            pltpu.VMEM((CH, H), jnp.float32),
            pltpu.SemaphoreType.DMA,
        ],
    )
    def k(t_hbm, src_hbm, out_hbm, sidx, buf, sem):
        wid = lax.axis_index("s") * NC + lax.axis_index("c")
        pltpu.sync_copy(src_hbm.at[wid], sidx)

        @pl.loop(0, NCH)
        def _(j):
            pltpu.async_copy(t_hbm.at[sidx.at[j]], buf, sem).wait()
            pltpu.sync_copy(buf, out_hbm.at[pl.ds(wid * EPW + j * CH, CH)])

    return k(table, src3)


def _sc_gather2(tableA, tableP, src3, dst3):
    @functools.partial(
        pl.kernel,
        out_type=jax.ShapeDtypeStruct((EP, H), jnp.float32),
        mesh=_sc_mesh(),
        scratch_types=[
            pltpu.VMEM((NCH, CH), jnp.int32),
            pltpu.VMEM((NCH, CH), jnp.int32),
            pltpu.VMEM((CH, H), jnp.float32),
            pltpu.SemaphoreType.DMA,
        ],
    )
    def k(a_hbm, p_hbm, src_hbm, dst_hbm, out_hbm, sidx, didx, buf, sem):
        wid = lax.axis_index("s") * NC + lax.axis_index("c")
        pltpu.sync_copy(src_hbm.at[wid], sidx)
        pltpu.sync_copy(dst_hbm.at[wid], didx)

        @pl.loop(0, NCH)
        def _(j):
            pltpu.async_copy(a_hbm.at[sidx.at[j]], buf, sem).wait()
            pltpu.async_copy(p_hbm.at[didx.at[j]], buf, sem, add=True).wait()
            pltpu.sync_copy(buf, out_hbm.at[pl.ds(wid * EPW + j * CH, CH)])

    return k(tableA, tableP, src3, dst3)


def _sc_scatter(messages, dst3, zeros_hbm):
    rows = PB // NS

    @functools.partial(
        pl.kernel,
        out_type=jax.ShapeDtypeStruct((NC, PB, H), jnp.float32),
        mesh=_sc_mesh(),
        scratch_types=[
            pltpu.VMEM((NCH, CH), jnp.int32),
            pltpu.VMEM((CH, H), jnp.float32),
            pltpu.VMEM_SHARED((PB, H), jnp.float32),
            pltpu.SemaphoreType.DMA,
        ],
    )
    def k(m_hbm, dst_hbm, z_hbm, out_hbm, didx, buf, acc, sem):
        c = lax.axis_index("c")
        s = lax.axis_index("s")
        wid = s * NC + c
        pltpu.sync_copy(dst_hbm.at[wid], didx)
        pltpu.sync_copy(z_hbm.at[pl.ds(s * rows, rows)],
                        acc.at[pl.ds(s * rows, rows)])
        plsc.subcore_barrier()

        @pl.loop(0, NCH)
        def _(j):
            pltpu.async_copy(m_hbm.at[pl.ds(wid * EPW + j * CH, CH)], buf,
                             sem).wait()
            pltpu.sync_copy(buf, acc.at[didx.at[j]], add=True)

        plsc.subcore_barrier()
        pltpu.sync_copy(acc.at[pl.ds(s * rows, rows)],
                        out_hbm.at[c, pl.ds(s * rows, rows)])

    return k(messages, dst3, zeros_hbm)


def kernel(atom_rep_0, atom_rep_1, atom_rep_2, probe_edges_features, params,
           probe_edges):
    f32 = jnp.float32
    src = probe_edges[:, 0].astype(jnp.int32)
    dst = probe_edges[:, 1].astype(jnp.int32)
    pad = EP - E
    src3 = jnp.concatenate([src, jnp.zeros((pad,), jnp.int32)]).reshape(NW, NCH, CH)
    dst3 = jnp.concatenate([dst, jnp.full((pad,), NP, jnp.int32)]).reshape(NW, NCH, CH)
    d_pad = jnp.concatenate(
        [probe_edges_features.astype(f32),
         jnp.full((pad, 1), 2.0 * CUTOFF, f32)], axis=0)

    waT, wbT, b1s, m2T, b2s, f1T, bf1, f2T, bf2 = [], [], [], [], [], [], [], [], []
    for i in range(NI):
        p = params["msg"][i]
        w1 = p["m1"][0]
        waT.append(w1[:, :H].T)
        wbT.append(w1[:, H:].T)
        b1s.append(p["m1"][1][None, :])
        m2T.append(p["m2"][0].T)
        b2s.append(p["m2"][1][None, :])
        f1T.append(jnp.zeros((ESP, H), f32).at[:ES].set(p["f1"][0].T))
        bf1.append(p["f1"][1][None, :])
        f2T.append(p["f2"][0].T)
        bf2.append(p["f2"][1][None, :])
    gw = [[params["gate"][i][k][0].T for k in ("g1", "g2")] for i in range(NI)]
    gb = [[params["gate"][i][k][1][None, :] for k in ("g1", "g2")] for i in range(NI)]
    sw = [[params["state"][i][k][0].T for k in ("s1", "s2")] for i in range(NI)]
    sb = [[params["state"][i][k][1][None, :] for k in ("s1", "s2")] for i in range(NI)]
    r1T = params["readout"]["r1"][0].T
    br1 = params["readout"]["r1"][1][None, :]
    r2T = jnp.zeros((H, 8), f32).at[:, 0].set(params["readout"]["r2"][0][0])
    br2 = jnp.zeros((1, 8), f32).at[0, 0].set(params["readout"]["r2"][1][0])

    mu = jnp.arange(0.0, CUTOFF, STEP, dtype=f32)
    mu_pad = jnp.full((1, ESP), 1.0e3, f32).at[0, :ES].set(mu)

    zeros_pb = jnp.zeros((PB, H), f32)

    atoms_s = jnp.stack([atom_rep_0, atom_rep_1, atom_rep_2])
    A_all = _precompute_A(atoms_s, jnp.stack(waT))

    probe = zeros_pb
    P = None
    for i in range(NI):
        if i == 0:
            hidden = _sc_gather1(A_all[0], src3)
        else:
            hidden = _sc_gather2(A_all[i], P, src3, dst3)
        messages = _edge_mlp(hidden, d_pad, b1s[i], m2T[i], b2s[i], f1T[i],
                             bf1[i], f2T[i], bf2[i], mu_pad)
        partials = _sc_scatter(messages, dst3, zeros_pb)
        gwi = [gw[i][0], gb[i][0], gw[i][1], gb[i][1]]
        swi = [sw[i][0], sb[i][0], sw[i][1], sb[i][1]]
        if i < NI - 1:
            probe, P = _probe_step(partials, probe, gwi, swi, wbT[i + 1])
        else:
            ro = _probe_final(partials, probe, gwi, swi, [r1T, br1, r2T, br2])
    return ro[:NP, 0][None, :]


# speedup vs baseline: 1.4979x; 1.0486x over previous
import functools
import math

import jax
import jax.numpy as jnp
from jax import lax
from jax.experimental import pallas as pl
from jax.experimental.pallas import tpu as pltpu
from jax.experimental.pallas import tpu_sc as plsc

H = 128
CUTOFF = 5.0
STEP = 0.1
ES = 50
ESP = 64
NI = 3
NA = 10000
NP = 8000
E = 128000
PB = 8192
NC = 2
NS = 16
NW = NC * NS
EP = 131072
EPW = EP // NW
CH = 128
NCH = EPW // CH
LOG2 = float(math.log(2.0))
BE = 2048
BP = 1024


def _ssp(x):
    return jnp.maximum(x, 0.0) + jnp.log1p(jnp.exp(-jnp.abs(x))) - LOG2


def _pre_body(a_ref, w_ref, o_ref):
    o_ref[0] = jnp.dot(a_ref[0], w_ref[0], preferred_element_type=jnp.float32)


def _precompute_A(atoms_s, waT_s):
    return pl.pallas_call(
        _pre_body,
        grid=(NI,),
        in_specs=[
            pl.BlockSpec((1, NA, H), lambda i: (i, 0, 0)),
            pl.BlockSpec((1, H, H), lambda i: (i, 0, 0)),
        ],
        out_specs=pl.BlockSpec((1, NA, H), lambda i: (i, 0, 0)),
        out_shape=jax.ShapeDtypeStruct((NI, NA, H), jnp.float32),
    )(atoms_s, waT_s)


def _edge_body(h_ref, d_ref, b1_ref, m2_ref, b2_ref, f1_ref, bf1_ref,
               f2_ref, bf2_ref, mu_ref, o_ref):
    d = d_ref[...]
    h = _ssp(h_ref[...] + b1_ref[...])
    m = jnp.dot(h, m2_ref[...], preferred_element_type=jnp.float32) + b2_ref[...]
    es = jnp.exp((-1.0 / (2.0 * STEP * STEP)) * (d - mu_ref[...]) ** 2)
    t = _ssp(jnp.dot(es, f1_ref[...], preferred_element_type=jnp.float32)
             + bf1_ref[...])
    fw = jnp.dot(t, f2_ref[...], preferred_element_type=jnp.float32) + bf2_ref[...]
    cc = jnp.where(d < CUTOFF, 0.5 * (jnp.cos((jnp.pi / CUTOFF) * d) + 1.0), 0.0)
    o_ref[...] = m * (fw * cc)


def _edge_mlp(hidden, d_pad, b1, m2T, b2, f1T, bf1, f2T, bf2, mu_pad):
    full = lambda r, c: pl.BlockSpec((r, c), lambda i: (0, 0))
    return pl.pallas_call(
        _edge_body,
        grid=(EP // BE,),
        in_specs=[
            pl.BlockSpec((BE, H), lambda i: (i, 0)),
            pl.BlockSpec((BE, 1), lambda i: (i, 0)),
            full(1, H), full(H, H), full(1, H),
            full(ESP, H), full(1, H), full(H, H), full(1, H), full(1, ESP),
        ],
        out_specs=pl.BlockSpec((BE, H), lambda i: (i, 0)),
        out_shape=jax.ShapeDtypeStruct((EP, H), jnp.float32),
    )(hidden, d_pad, b1, m2T, b2, f1T, bf1, f2T, bf2, mu_pad)


def _probe_common(p_ref, ps_ref, g1, bg1, g2, bg2, s1, bs1, s2, bs2):
    msgsum = p_ref[0] + p_ref[1]
    probe = ps_ref[...]
    gates = jax.nn.sigmoid(
        jnp.dot(_ssp(jnp.dot(probe, g1[...], preferred_element_type=jnp.float32)
                     + bg1[...]), g2[...], preferred_element_type=jnp.float32)
        + bg2[...])
    trans = (jnp.dot(_ssp(jnp.dot(msgsum, s1[...],
                                  preferred_element_type=jnp.float32) + bs1[...]),
                     s2[...], preferred_element_type=jnp.float32) + bs2[...])
    return probe * gates + (1.0 - gates) * trans


def _probe_step_body(p_ref, ps_ref, g1, bg1, g2, bg2, s1, bs1, s2, bs2, wbn,
                     np_ref, pn_ref):
    newp = _probe_common(p_ref, ps_ref, g1, bg1, g2, bg2, s1, bs1, s2, bs2)
    np_ref[...] = newp
    pn_ref[...] = jnp.dot(newp, wbn[...], preferred_element_type=jnp.float32)


def _probe_final_body(p_ref, ps_ref, g1, bg1, g2, bg2, s1, bs1, s2, bs2,
                      r1, br1, r2, br2, o_ref):
    newp = _probe_common(p_ref, ps_ref, g1, bg1, g2, bg2, s1, bs1, s2, bs2)
    ro = jnp.dot(_ssp(jnp.dot(newp, r1[...], preferred_element_type=jnp.float32)
                      + br1[...]), r2[...], preferred_element_type=jnp.float32)
    o_ref[...] = ro + br2[...]


def _probe_step(partials, probe, gw, sw, wbnT):
    full = lambda r, c: pl.BlockSpec((r, c), lambda i: (0, 0))
    row = pl.BlockSpec((BP, H), lambda i: (i, 0))
    return pl.pallas_call(
        _probe_step_body,
        grid=(PB // BP,),
        in_specs=[
            pl.BlockSpec((NC, BP, H), lambda i: (0, i, 0)), row,
            full(H, H), full(1, H), full(H, H), full(1, H),
            full(H, H), full(1, H), full(H, H), full(1, H),
            full(H, H),
        ],
        out_specs=[row, row],
        out_shape=[jax.ShapeDtypeStruct((PB, H), jnp.float32),
                   jax.ShapeDtypeStruct((PB, H), jnp.float32)],
    )(partials, probe, *gw, *sw, wbnT)


def _probe_final(partials, probe, gw, sw, rw):
    full = lambda r, c: pl.BlockSpec((r, c), lambda i: (0, 0))
    row = pl.BlockSpec((BP, H), lambda i: (i, 0))
    return pl.pallas_call(
        _probe_final_body,
        grid=(PB // BP,),
        in_specs=[
            pl.BlockSpec((NC, BP, H), lambda i: (0, i, 0)), row,
            full(H, H), full(1, H), full(H, H), full(1, H),
            full(H, H), full(1, H), full(H, H), full(1, H),
            full(H, H), full(1, H), full(H, 8), full(1, 8),
        ],
        out_specs=pl.BlockSpec((BP, 8), lambda i: (i, 0)),
        out_shape=jax.ShapeDtypeStruct((PB, 8), jnp.float32),
    )(partials, probe, *gw, *sw, *rw)


def _sc_mesh():
    return plsc.VectorSubcoreMesh(core_axis_name="c", subcore_axis_name="s",
                                  num_cores=NC)


KB = 4


def _sc_gather1(table, src3):
    @functools.partial(
        pl.kernel,
        out_type=jax.ShapeDtypeStruct((EP, H), jnp.float32),
        mesh=_sc_mesh(),
        scratch_types=(
            [pltpu.VMEM((NCH, CH), jnp.int32)]
            + [pltpu.VMEM((CH, H), jnp.float32) for _ in range(KB)]
            + [pltpu.SemaphoreType.DMA for _ in range(KB)]
        ),
    )
    def k(t_hbm, src_hbm, out_hbm, sidx, *rest):
        bufs, sems = rest[:KB], rest[KB:]
        wid = lax.axis_index("s") * NC + lax.axis_index("c")
        base = wid * EPW
        pltpu.sync_copy(src_hbm.at[wid], sidx)

        @pl.loop(0, NCH // KB)
        def _(g):
            j0 = g * KB
            dA = [pltpu.async_copy(t_hbm.at[sidx.at[j0 + b]], bufs[b], sems[b])
                  for b in range(KB)]
            dW = []
            for b in range(KB):
                dA[b].wait()
                dW.append(pltpu.async_copy(
                    bufs[b], out_hbm.at[pl.ds(base + (j0 + b) * CH, CH)],
                    sems[b]))
            for b in range(KB):
                dW[b].wait()

    return k(table, src3)


def _sc_gather2(tableA, tableP, src3, dst3):
    @functools.partial(
        pl.kernel,
        out_type=jax.ShapeDtypeStruct((EP, H), jnp.float32),
        mesh=_sc_mesh(),
        scratch_types=(
            [pltpu.VMEM((NCH, CH), jnp.int32),
             pltpu.VMEM((NCH, CH), jnp.int32)]
            + [pltpu.VMEM((CH, H), jnp.float32) for _ in range(KB)]
            + [pltpu.SemaphoreType.DMA for _ in range(KB)]
        ),
    )
    def k(a_hbm, p_hbm, src_hbm, dst_hbm, out_hbm, sidx, didx, *rest):
        bufs, sems = rest[:KB], rest[KB:]
        wid = lax.axis_index("s") * NC + lax.axis_index("c")
        base = wid * EPW
        pltpu.sync_copy(src_hbm.at[wid], sidx)
        pltpu.sync_copy(dst_hbm.at[wid], didx)

        @pl.loop(0, NCH // KB)
        def _(g):
            j0 = g * KB
            dA = [pltpu.async_copy(a_hbm.at[sidx.at[j0 + b]], bufs[b], sems[b])
                  for b in range(KB)]
            dP = []
            for b in range(KB):
                dA[b].wait()
                dP.append(pltpu.async_copy(p_hbm.at[didx.at[j0 + b]], bufs[b],
                                           sems[b], add=True))
            dW = []
            for b in range(KB):
                dP[b].wait()
                dW.append(pltpu.async_copy(
                    bufs[b], out_hbm.at[pl.ds(base + (j0 + b) * CH, CH)],
                    sems[b]))
            for b in range(KB):
                dW[b].wait()

    return k(tableA, tableP, src3, dst3)


def _sc_scatter(messages, dst3, zeros_hbm):
    rows = PB // NS
    KS = 2

    @functools.partial(
        pl.kernel,
        out_type=jax.ShapeDtypeStruct((NC, PB, H), jnp.float32),
        mesh=_sc_mesh(),
        scratch_types=(
            [pltpu.VMEM((NCH, CH), jnp.int32),
             pltpu.VMEM_SHARED((PB, H), jnp.float32)]
            + [pltpu.VMEM((CH, H), jnp.float32) for _ in range(KS)]
            + [pltpu.SemaphoreType.DMA for _ in range(KS)]
        ),
    )
    def k(m_hbm, dst_hbm, z_hbm, out_hbm, didx, acc, *rest):
        bufs, sems = rest[:KS], rest[KS:]
        c = lax.axis_index("c")
        s = lax.axis_index("s")
        wid = s * NC + c
        base = wid * EPW
        pltpu.sync_copy(dst_hbm.at[wid], didx)
        pltpu.sync_copy(z_hbm.at[pl.ds(s * rows, rows)],
                        acc.at[pl.ds(s * rows, rows)])
        plsc.subcore_barrier()

        @pl.loop(0, NCH // KS)
        def _(g):
            j0 = g * KS
            dR = [pltpu.async_copy(
                      m_hbm.at[pl.ds(base + (j0 + b) * CH, CH)], bufs[b],
                      sems[b]) for b in range(KS)]
            dS = []
            for b in range(KS):
                dR[b].wait()
                dS.append(pltpu.async_copy(bufs[b], acc.at[didx.at[j0 + b]],
                                           sems[b], add=True))
            for b in range(KS):
                dS[b].wait()

        plsc.subcore_barrier()
        pltpu.sync_copy(acc.at[pl.ds(s * rows, rows)],
                        out_hbm.at[c, pl.ds(s * rows, rows)])

    return k(messages, dst3, zeros_hbm)


def kernel(atom_rep_0, atom_rep_1, atom_rep_2, probe_edges_features, params,
           probe_edges):
    f32 = jnp.float32
    src = probe_edges[:, 0].astype(jnp.int32)
    dst = probe_edges[:, 1].astype(jnp.int32)
    pad = EP - E
    src3 = jnp.concatenate([src, jnp.zeros((pad,), jnp.int32)]).reshape(NW, NCH, CH)
    dst3 = jnp.concatenate([dst, jnp.full((pad,), NP, jnp.int32)]).reshape(NW, NCH, CH)
    d_pad = jnp.concatenate(
        [probe_edges_features.astype(f32),
         jnp.full((pad, 1), 2.0 * CUTOFF, f32)], axis=0)

    waT, wbT, b1s, m2T, b2s, f1T, bf1, f2T, bf2 = [], [], [], [], [], [], [], [], []
    for i in range(NI):
        p = params["msg"][i]
        w1 = p["m1"][0]
        waT.append(w1[:, :H].T)
        wbT.append(w1[:, H:].T)
        b1s.append(p["m1"][1][None, :])
        m2T.append(p["m2"][0].T)
        b2s.append(p["m2"][1][None, :])
        f1T.append(jnp.zeros((ESP, H), f32).at[:ES].set(p["f1"][0].T))
        bf1.append(p["f1"][1][None, :])
        f2T.append(p["f2"][0].T)
        bf2.append(p["f2"][1][None, :])
    gw = [[params["gate"][i][k][0].T for k in ("g1", "g2")] for i in range(NI)]
    gb = [[params["gate"][i][k][1][None, :] for k in ("g1", "g2")] for i in range(NI)]
    sw = [[params["state"][i][k][0].T for k in ("s1", "s2")] for i in range(NI)]
    sb = [[params["state"][i][k][1][None, :] for k in ("s1", "s2")] for i in range(NI)]
    r1T = params["readout"]["r1"][0].T
    br1 = params["readout"]["r1"][1][None, :]
    r2T = jnp.zeros((H, 8), f32).at[:, 0].set(params["readout"]["r2"][0][0])
    br2 = jnp.zeros((1, 8), f32).at[0, 0].set(params["readout"]["r2"][1][0])

    mu = jnp.arange(0.0, CUTOFF, STEP, dtype=f32)
    mu_pad = jnp.full((1, ESP), 1.0e3, f32).at[0, :ES].set(mu)

    zeros_pb = jnp.zeros((PB, H), f32)

    atoms_s = jnp.stack([atom_rep_0, atom_rep_1, atom_rep_2])
    A_all = _precompute_A(atoms_s, jnp.stack(waT))

    probe = zeros_pb
    P = None
    for i in range(NI):
        if i == 0:
            hidden = _sc_gather1(A_all[0], src3)
        else:
            hidden = _sc_gather2(A_all[i], P, src3, dst3)
        messages = _edge_mlp(hidden, d_pad, b1s[i], m2T[i], b2s[i], f1T[i],
                             bf1[i], f2T[i], bf2[i], mu_pad)
        partials = _sc_scatter(messages, dst3, zeros_pb)
        gwi = [gw[i][0], gb[i][0], gw[i][1], gb[i][1]]
        swi = [sw[i][0], sb[i][0], sw[i][1], sb[i][1]]
        if i < NI - 1:
            probe, P = _probe_step(partials, probe, gwi, swi, wbT[i + 1])
        else:
            ro = _probe_final(partials, probe, gwi, swi, [r1T, br1, r2T, br2])
    return ro[:NP, 0][None, :]


# speedup vs baseline: 1.8787x; 1.2542x over previous
import functools
import math

import jax
import jax.numpy as jnp
from jax import lax
from jax.experimental import pallas as pl
from jax.experimental.pallas import tpu as pltpu
from jax.experimental.pallas import tpu_sc as plsc

H = 128
CUTOFF = 5.0
STEP = 0.1
ES = 50
ESP = 64
NI = 3
NA = 10000
NAP = 10240
NP = 8000
E = 128000
PB = 8192
NC = 2
NS = 16
NW = NC * NS
EP = 131072
EPW = EP // NW
CH = 128
NCH = EPW // CH
LOG2 = float(math.log(2.0))
BE = 2048
BP = 1024


def _ssp(x):
    return jnp.maximum(x, 0.0) + jnp.log1p(jnp.exp(-jnp.abs(x))) - LOG2


def _pre_body(a_ref, w_ref, o_ref):
    o_ref[0] = jnp.dot(a_ref[0], w_ref[0], preferred_element_type=jnp.float32)


def _precompute_A(atoms_s, waT_s):
    return pl.pallas_call(
        _pre_body,
        grid=(NI,),
        in_specs=[
            pl.BlockSpec((1, NAP, H), lambda i: (i, 0, 0)),
            pl.BlockSpec((1, H, H), lambda i: (i, 0, 0)),
        ],
        out_specs=pl.BlockSpec((1, NAP, H), lambda i: (i, 0, 0)),
        out_shape=jax.ShapeDtypeStruct((NI, NA, H), jnp.float32),
    )(atoms_s, waT_s)


def _edge_body(h_ref, d_ref, b1_ref, m2_ref, b2_ref, f1_ref, bf1_ref,
               f2_ref, bf2_ref, mu_ref, o_ref):
    d = d_ref[...]
    h = _ssp(h_ref[...] + b1_ref[...])
    m = jnp.dot(h, m2_ref[...], preferred_element_type=jnp.float32) + b2_ref[...]
    es = jnp.exp((-1.0 / (2.0 * STEP * STEP)) * (d - mu_ref[...]) ** 2)
    t = _ssp(jnp.dot(es, f1_ref[...], preferred_element_type=jnp.float32)
             + bf1_ref[...])
    fw = jnp.dot(t, f2_ref[...], preferred_element_type=jnp.float32) + bf2_ref[...]
    cc = jnp.where(d < CUTOFF, 0.5 * (jnp.cos((jnp.pi / CUTOFF) * d) + 1.0), 0.0)
    o_ref[...] = m * (fw * cc)


def _edge_mlp(hidden, d_pad, b1, m2T, b2, f1T, bf1, f2T, bf2, mu_pad):
    full = lambda r, c: pl.BlockSpec((r, c), lambda i: (0, 0))
    return pl.pallas_call(
        _edge_body,
        grid=(EP // BE,),
        in_specs=[
            pl.BlockSpec((BE, H), lambda i: (i, 0)),
            pl.BlockSpec((BE, 1), lambda i: (i, 0)),
            full(1, H), full(H, H), full(1, H),
            full(ESP, H), full(1, H), full(H, H), full(1, H), full(1, ESP),
        ],
        out_specs=pl.BlockSpec((BE, H), lambda i: (i, 0)),
        out_shape=jax.ShapeDtypeStruct((EP, H), jnp.float32),
    )(hidden, d_pad, b1, m2T, b2, f1T, bf1, f2T, bf2, mu_pad)


def _probe_common(p_ref, ps_ref, g1, bg1, g2, bg2, s1, bs1, s2, bs2):
    msgsum = p_ref[0] + p_ref[1]
    probe = ps_ref[...]
    gates = jax.nn.sigmoid(
        jnp.dot(_ssp(jnp.dot(probe, g1[...], preferred_element_type=jnp.float32)
                     + bg1[...]), g2[...], preferred_element_type=jnp.float32)
        + bg2[...])
    trans = (jnp.dot(_ssp(jnp.dot(msgsum, s1[...],
                                  preferred_element_type=jnp.float32) + bs1[...]),
                     s2[...], preferred_element_type=jnp.float32) + bs2[...])
    return probe * gates + (1.0 - gates) * trans


def _probe_step_body(p_ref, ps_ref, g1, bg1, g2, bg2, s1, bs1, s2, bs2, wbn,
                     np_ref, pn_ref):
    newp = _probe_common(p_ref, ps_ref, g1, bg1, g2, bg2, s1, bs1, s2, bs2)
    np_ref[...] = newp
    pn_ref[...] = jnp.dot(newp, wbn[...], preferred_element_type=jnp.float32)


def _probe_final_body(p_ref, ps_ref, g1, bg1, g2, bg2, s1, bs1, s2, bs2,
                      r1, br1, r2, br2, o_ref):
    newp = _probe_common(p_ref, ps_ref, g1, bg1, g2, bg2, s1, bs1, s2, bs2)
    ro = jnp.dot(_ssp(jnp.dot(newp, r1[...], preferred_element_type=jnp.float32)
                      + br1[...]), r2[...], preferred_element_type=jnp.float32)
    o_ref[...] = ro + br2[...]


def _probe_step(partials, probe, gw, sw, wbnT):
    full = lambda r, c: pl.BlockSpec((r, c), lambda i: (0, 0))
    row = pl.BlockSpec((BP, H), lambda i: (i, 0))
    return pl.pallas_call(
        _probe_step_body,
        grid=(PB // BP,),
        in_specs=[
            pl.BlockSpec((NC, BP, H), lambda i: (0, i, 0)), row,
            full(H, H), full(1, H), full(H, H), full(1, H),
            full(H, H), full(1, H), full(H, H), full(1, H),
            full(H, H),
        ],
        out_specs=[row, row],
        out_shape=[jax.ShapeDtypeStruct((PB, H), jnp.float32),
                   jax.ShapeDtypeStruct((PB, H), jnp.float32)],
    )(partials, probe, *gw, *sw, wbnT)


def _probe_final(partials, probe, gw, sw, rw):
    full = lambda r, c: pl.BlockSpec((r, c), lambda i: (0, 0))
    row = pl.BlockSpec((BP, H), lambda i: (i, 0))
    return pl.pallas_call(
        _probe_final_body,
        grid=(PB // BP,),
        in_specs=[
            pl.BlockSpec((NC, BP, H), lambda i: (0, i, 0)), row,
            full(H, H), full(1, H), full(H, H), full(1, H),
            full(H, H), full(1, H), full(H, H), full(1, H),
            full(H, H), full(1, H), full(H, 8), full(1, 8),
        ],
        out_specs=pl.BlockSpec((BP, 8), lambda i: (i, 0)),
        out_shape=jax.ShapeDtypeStruct((PB, 8), jnp.float32),
    )(partials, probe, *gw, *sw, *rw)


def _sc_mesh():
    return plsc.VectorSubcoreMesh(core_axis_name="c", subcore_axis_name="s",
                                  num_cores=NC)


def _sc_gather1(table, src3):
    KB = 2
    trows = NAP // NS

    @functools.partial(
        pl.kernel,
        out_type=jax.ShapeDtypeStruct((EP, H), jnp.float32),
        mesh=_sc_mesh(),
        scratch_types=(
            [pltpu.VMEM((NCH, CH), jnp.int32),
             pltpu.VMEM_SHARED((NAP, H), jnp.float32)]
            + [pltpu.VMEM((CH, H), jnp.float32) for _ in range(KB)]
            + [pltpu.SemaphoreType.DMA for _ in range(KB)]
        ),
    )
    def k(t_hbm, src_hbm, out_hbm, sidx, tsp, *rest):
        bufs, sems = rest[:KB], rest[KB:]
        c = lax.axis_index("c")
        s = lax.axis_index("s")
        wid = s * NC + c
        base = wid * EPW
        pltpu.sync_copy(src_hbm.at[wid], sidx)
        pltpu.sync_copy(t_hbm.at[pl.ds(s * trows, trows)],
                        tsp.at[pl.ds(s * trows, trows)])
        plsc.subcore_barrier()

        @pl.loop(0, NCH // KB)
        def _(g):
            j0 = g * KB
            dA = [pltpu.async_copy(tsp.at[sidx.at[j0 + b]], bufs[b], sems[b])
                  for b in range(KB)]
            dW = []
            for b in range(KB):
                dA[b].wait()
                dW.append(pltpu.async_copy(
                    bufs[b], out_hbm.at[pl.ds(base + (j0 + b) * CH, CH)],
                    sems[b]))
            for b in range(KB):
                dW[b].wait()

    return k(table, src3)


def _sc_gather2(tableA, tableP, src3, dst3):
    KB = 2
    prows = PB // NS

    @functools.partial(
        pl.kernel,
        out_type=jax.ShapeDtypeStruct((EP, H), jnp.float32),
        mesh=_sc_mesh(),
        scratch_types=(
            [pltpu.VMEM((NCH, CH), jnp.int32),
             pltpu.VMEM((NCH, CH), jnp.int32),
             pltpu.VMEM_SHARED((PB, H), jnp.float32)]
            + [pltpu.VMEM((CH, H), jnp.float32) for _ in range(KB)]
            + [pltpu.SemaphoreType.DMA for _ in range(KB)]
        ),
    )
    def k(a_hbm, p_hbm, src_hbm, dst_hbm, out_hbm, sidx, didx, psp, *rest):
        bufs, sems = rest[:KB], rest[KB:]
        c = lax.axis_index("c")
        s = lax.axis_index("s")
        wid = s * NC + c
        base = wid * EPW
        pltpu.sync_copy(src_hbm.at[wid], sidx)
        pltpu.sync_copy(dst_hbm.at[wid], didx)
        pltpu.sync_copy(p_hbm.at[pl.ds(s * prows, prows)],
                        psp.at[pl.ds(s * prows, prows)])
        plsc.subcore_barrier()

        @pl.loop(0, NCH // KB)
        def _(g):
            j0 = g * KB
            dA = [pltpu.async_copy(a_hbm.at[sidx.at[j0 + b]], bufs[b], sems[b])
                  for b in range(KB)]
            dP = []
            for b in range(KB):
                dA[b].wait()
                dP.append(pltpu.async_copy(psp.at[didx.at[j0 + b]], bufs[b],
                                           sems[b], add=True))
            dW = []
            for b in range(KB):
                dP[b].wait()
                dW.append(pltpu.async_copy(
                    bufs[b], out_hbm.at[pl.ds(base + (j0 + b) * CH, CH)],
                    sems[b]))
            for b in range(KB):
                dW[b].wait()

    return k(tableA, tableP, src3, dst3)


def _sc_scatter(messages, dst3, zeros_hbm):
    rows = PB // NS
    KS = 2

    @functools.partial(
        pl.kernel,
        out_type=jax.ShapeDtypeStruct((NC, PB, H), jnp.float32),
        mesh=_sc_mesh(),
        scratch_types=(
            [pltpu.VMEM((NCH, CH), jnp.int32),
             pltpu.VMEM_SHARED((PB, H), jnp.float32)]
            + [pltpu.VMEM((CH, H), jnp.float32) for _ in range(KS)]
            + [pltpu.SemaphoreType.DMA for _ in range(KS)]
        ),
    )
    def k(m_hbm, dst_hbm, z_hbm, out_hbm, didx, acc, *rest):
        bufs, sems = rest[:KS], rest[KS:]
        c = lax.axis_index("c")
        s = lax.axis_index("s")
        wid = s * NC + c
        base = wid * EPW
        pltpu.sync_copy(dst_hbm.at[wid], didx)
        pltpu.sync_copy(z_hbm.at[pl.ds(s * rows, rows)],
                        acc.at[pl.ds(s * rows, rows)])
        plsc.subcore_barrier()

        @pl.loop(0, NCH // KS)
        def _(g):
            j0 = g * KS
            dR = [pltpu.async_copy(
                      m_hbm.at[pl.ds(base + (j0 + b) * CH, CH)], bufs[b],
                      sems[b]) for b in range(KS)]
            dS = []
            for b in range(KS):
                dR[b].wait()
                dS.append(pltpu.async_copy(bufs[b], acc.at[didx.at[j0 + b]],
                                           sems[b], add=True))
            for b in range(KS):
                dS[b].wait()

        plsc.subcore_barrier()
        pltpu.sync_copy(acc.at[pl.ds(s * rows, rows)],
                        out_hbm.at[c, pl.ds(s * rows, rows)])

    return k(messages, dst3, zeros_hbm)


def kernel(atom_rep_0, atom_rep_1, atom_rep_2, probe_edges_features, params,
           probe_edges):
    f32 = jnp.float32
    src = probe_edges[:, 0].astype(jnp.int32)
    dst = probe_edges[:, 1].astype(jnp.int32)
    pad = EP - E
    src3 = jnp.concatenate([src, jnp.zeros((pad,), jnp.int32)]).reshape(NW, NCH, CH)
    dst3 = jnp.concatenate([dst, jnp.full((pad,), NP, jnp.int32)]).reshape(NW, NCH, CH)
    d_pad = jnp.concatenate(
        [probe_edges_features.astype(f32),
         jnp.full((pad, 1), 2.0 * CUTOFF, f32)], axis=0)

    waT, wbT, b1s, m2T, b2s, f1T, bf1, f2T, bf2 = [], [], [], [], [], [], [], [], []
    for i in range(NI):
        p = params["msg"][i]
        w1 = p["m1"][0]
        waT.append(w1[:, :H].T)
        wbT.append(w1[:, H:].T)
        b1s.append(p["m1"][1][None, :])
        m2T.append(p["m2"][0].T)
        b2s.append(p["m2"][1][None, :])
        f1T.append(jnp.zeros((ESP, H), f32).at[:ES].set(p["f1"][0].T))
        bf1.append(p["f1"][1][None, :])
        f2T.append(p["f2"][0].T)
        bf2.append(p["f2"][1][None, :])
    gw = [[params["gate"][i][k][0].T for k in ("g1", "g2")] for i in range(NI)]
    gb = [[params["gate"][i][k][1][None, :] for k in ("g1", "g2")] for i in range(NI)]
    sw = [[params["state"][i][k][0].T for k in ("s1", "s2")] for i in range(NI)]
    sb = [[params["state"][i][k][1][None, :] for k in ("s1", "s2")] for i in range(NI)]
    r1T = params["readout"]["r1"][0].T
    br1 = params["readout"]["r1"][1][None, :]
    r2T = jnp.zeros((H, 8), f32).at[:, 0].set(params["readout"]["r2"][0][0])
    br2 = jnp.zeros((1, 8), f32).at[0, 0].set(params["readout"]["r2"][1][0])

    mu = jnp.arange(0.0, CUTOFF, STEP, dtype=f32)
    mu_pad = jnp.full((1, ESP), 1.0e3, f32).at[0, :ES].set(mu)

    zeros_pb = jnp.zeros((PB, H), f32)

    atoms_s = jnp.zeros((NI, NAP, H), f32).at[:, :NA].set(
        jnp.stack([atom_rep_0, atom_rep_1, atom_rep_2]))
    A_all = _precompute_A(atoms_s, jnp.stack(waT))

    probe = zeros_pb
    P = None
    for i in range(NI):
        if i == 0:
            hidden = _sc_gather1(A_all[0], src3)
        else:
            hidden = _sc_gather2(A_all[i], P, src3, dst3)
        messages = _edge_mlp(hidden, d_pad, b1s[i], m2T[i], b2s[i], f1T[i],
                             bf1[i], f2T[i], bf2[i], mu_pad)
        partials = _sc_scatter(messages, dst3, zeros_pb)
        gwi = [gw[i][0], gb[i][0], gw[i][1], gb[i][1]]
        swi = [sw[i][0], sb[i][0], sw[i][1], sb[i][1]]
        if i < NI - 1:
            probe, P = _probe_step(partials, probe, gwi, swi, wbT[i + 1])
        else:
            ro = _probe_final(partials, probe, gwi, swi, [r1T, br1, r2T, br2])
    return ro[:NP, 0][None, :]


# speedup vs baseline: 2.6126x; 1.3907x over previous
import functools
import math

import jax
import jax.numpy as jnp
from jax import lax
from jax.experimental import pallas as pl
from jax.experimental.pallas import tpu as pltpu
from jax.experimental.pallas import tpu_sc as plsc

H = 128
CUTOFF = 5.0
STEP = 0.1
ES = 50
ESP = 64
NI = 3
NA = 10000
NAP = 10240
NP = 8000
E = 128000
PB = 8192
NC = 2
NS = 16
NW = NC * NS
EP = 131072
EPW = EP // NW
CH = 128
NCH = EPW // CH
LOG2 = float(math.log(2.0))
BE = 2048
BP = 1024


def _ssp(x):
    return jnp.maximum(x, 0.0) + jnp.log1p(jnp.exp(-jnp.abs(x))) - LOG2


def _pre_body(a_ref, w_ref, o_ref):
    o_ref[0] = jnp.dot(a_ref[0], w_ref[0], preferred_element_type=jnp.float32)


def _precompute_A(atoms_s, waT_s):
    return pl.pallas_call(
        _pre_body,
        grid=(NI,),
        in_specs=[
            pl.BlockSpec((1, NAP, H), lambda i: (i, 0, 0)),
            pl.BlockSpec((1, H, H), lambda i: (i, 0, 0)),
        ],
        out_specs=pl.BlockSpec((1, NAP, H), lambda i: (i, 0, 0)),
        out_shape=jax.ShapeDtypeStruct((NI, NAP, H), jnp.float32),
    )(atoms_s, waT_s)


def _cc_body(d_ref, o_ref):
    d = d_ref[...]
    o_ref[...] = jnp.where(
        d < CUTOFF, 0.5 * (jnp.cos((jnp.pi / CUTOFF) * d) + 1.0), 0.0)


def _cc_precompute(d_sq):
    return pl.pallas_call(
        _cc_body,
        out_shape=jax.ShapeDtypeStruct((EP // H, H), jnp.float32),
    )(d_sq)


def _edge_body(h_ref, d_ref, cc_ref, ones_ref, b1_ref, m2_ref, b2_ref,
               f1_ref, bf1_ref, f2_ref, bf2_ref, mu_ref, o_ref):
    ccH = jnp.dot(cc_ref[...], ones_ref[...],
                  preferred_element_type=jnp.float32)
    dE = jnp.dot(d_ref[...], ones_ref[:, :ESP],
                 preferred_element_type=jnp.float32)
    h = _ssp(h_ref[...] + b1_ref[...])
    m = jnp.dot(h, m2_ref[...], preferred_element_type=jnp.float32) + b2_ref[...]
    es = jnp.exp((-1.0 / (2.0 * STEP * STEP)) * (dE - mu_ref[...]) ** 2)
    t = _ssp(jnp.dot(es, f1_ref[...], preferred_element_type=jnp.float32)
             + bf1_ref[...])
    fw = jnp.dot(t, f2_ref[...], preferred_element_type=jnp.float32) + bf2_ref[...]
    o_ref[...] = m * (fw * ccH)


def _edge_mlp(hidden, d_pad, cc_col, ones_row, b1, m2T, b2, f1T, bf1, f2T,
              bf2, mu_pad):
    full = lambda r, c: pl.BlockSpec((r, c), lambda i: (0, 0))
    return pl.pallas_call(
        _edge_body,
        grid=(EP // BE,),
        in_specs=[
            pl.BlockSpec((BE, H), lambda i: (i, 0)),
            pl.BlockSpec((BE, 1), lambda i: (i, 0)),
            pl.BlockSpec((BE, 1), lambda i: (i, 0)),
            full(1, H),
            full(1, H), full(H, H), full(1, H),
            full(ESP, H), full(1, H), full(H, H), full(1, H), full(1, ESP),
        ],
        out_specs=pl.BlockSpec((BE, H), lambda i: (i, 0)),
        out_shape=jax.ShapeDtypeStruct((EP, H), jnp.float32),
    )(hidden, d_pad, cc_col, ones_row, b1, m2T, b2, f1T, bf1, f2T, bf2,
      mu_pad)


def _probe_common(p_ref, ps_ref, g1, bg1, g2, bg2, s1, bs1, s2, bs2):
    msgsum = p_ref[0] + p_ref[1]
    probe = ps_ref[...]
    gates = jax.nn.sigmoid(
        jnp.dot(_ssp(jnp.dot(probe, g1[...], preferred_element_type=jnp.float32)
                     + bg1[...]), g2[...], preferred_element_type=jnp.float32)
        + bg2[...])
    trans = (jnp.dot(_ssp(jnp.dot(msgsum, s1[...],
                                  preferred_element_type=jnp.float32) + bs1[...]),
                     s2[...], preferred_element_type=jnp.float32) + bs2[...])
    return probe * gates + (1.0 - gates) * trans


def _probe_step_body(p_ref, ps_ref, g1, bg1, g2, bg2, s1, bs1, s2, bs2, wbn,
                     np_ref, pn_ref):
    newp = _probe_common(p_ref, ps_ref, g1, bg1, g2, bg2, s1, bs1, s2, bs2)
    np_ref[...] = newp
    pn_ref[...] = jnp.dot(newp, wbn[...], preferred_element_type=jnp.float32)


def _probe_final_body(p_ref, ps_ref, g1, bg1, g2, bg2, s1, bs1, s2, bs2,
                      r1, br1, r2, br2, o_ref):
    newp = _probe_common(p_ref, ps_ref, g1, bg1, g2, bg2, s1, bs1, s2, bs2)
    ro = jnp.dot(_ssp(jnp.dot(newp, r1[...], preferred_element_type=jnp.float32)
                      + br1[...]), r2[...], preferred_element_type=jnp.float32)
    o_ref[...] = ro + br2[...]


def _probe_step(partials, probe, gw, sw, wbnT):
    full = lambda r, c: pl.BlockSpec((r, c), lambda i: (0, 0))
    row = pl.BlockSpec((BP, H), lambda i: (i, 0))
    return pl.pallas_call(
        _probe_step_body,
        grid=(PB // BP,),
        in_specs=[
            pl.BlockSpec((NC, BP, H), lambda i: (0, i, 0)), row,
            full(H, H), full(1, H), full(H, H), full(1, H),
            full(H, H), full(1, H), full(H, H), full(1, H),
            full(H, H),
        ],
        out_specs=[row, row],
        out_shape=[jax.ShapeDtypeStruct((PB, H), jnp.float32),
                   jax.ShapeDtypeStruct((PB, H), jnp.float32)],
    )(partials, probe, *gw, *sw, wbnT)


def _probe_final(partials, probe, gw, sw, rw):
    full = lambda r, c: pl.BlockSpec((r, c), lambda i: (0, 0))
    row = pl.BlockSpec((BP, H), lambda i: (i, 0))
    return pl.pallas_call(
        _probe_final_body,
        grid=(PB // BP,),
        in_specs=[
            pl.BlockSpec((NC, BP, H), lambda i: (0, i, 0)), row,
            full(H, H), full(1, H), full(H, H), full(1, H),
            full(H, H), full(1, H), full(H, H), full(1, H),
            full(H, H), full(1, H), full(H, 8), full(1, 8),
        ],
        out_specs=pl.BlockSpec((BP, 8), lambda i: (i, 0)),
        out_shape=jax.ShapeDtypeStruct((PB, 8), jnp.float32),
    )(partials, probe, *gw, *sw, *rw)


def _sc_mesh():
    return plsc.VectorSubcoreMesh(core_axis_name="c", subcore_axis_name="s",
                                  num_cores=NC)


def _sc_gather1(table, src3):
    KB = 2
    trows = NAP // NS

    @functools.partial(
        pl.kernel,
        out_type=jax.ShapeDtypeStruct((EP, H), jnp.float32),
        mesh=_sc_mesh(),
        scratch_types=(
            [pltpu.VMEM((NCH, CH), jnp.int32),
             pltpu.VMEM_SHARED((NAP, H), jnp.float32)]
            + [pltpu.VMEM((CH, H), jnp.float32) for _ in range(KB)]
            + [pltpu.SemaphoreType.DMA for _ in range(KB)]
        ),
    )
    def k(t_hbm, src_hbm, out_hbm, sidx, tsp, *rest):
        bufs, sems = rest[:KB], rest[KB:]
        c = lax.axis_index("c")
        s = lax.axis_index("s")
        wid = s * NC + c
        base = wid * EPW
        pltpu.sync_copy(src_hbm.at[wid], sidx)
        pltpu.sync_copy(t_hbm.at[pl.ds(s * trows, trows)],
                        tsp.at[pl.ds(s * trows, trows)])
        plsc.subcore_barrier()

        @pl.loop(0, NCH // KB)
        def _(g):
            j0 = g * KB
            dA = [pltpu.async_copy(tsp.at[sidx.at[j0 + b]], bufs[b], sems[b])
                  for b in range(KB)]
            dW = []
            for b in range(KB):
                dA[b].wait()
                dW.append(pltpu.async_copy(
                    bufs[b], out_hbm.at[pl.ds(base + (j0 + b) * CH, CH)],
                    sems[b]))
            for b in range(KB):
                dW[b].wait()

    return k(table, src3)


def _sc_gather2(tableA, tableP, src3, dst3):
    KB = 2
    prows = PB // NS

    @functools.partial(
        pl.kernel,
        out_type=jax.ShapeDtypeStruct((EP, H), jnp.float32),
        mesh=_sc_mesh(),
        scratch_types=(
            [pltpu.VMEM((NCH, CH), jnp.int32),
             pltpu.VMEM((NCH, CH), jnp.int32),
             pltpu.VMEM_SHARED((PB, H), jnp.float32)]
            + [pltpu.VMEM((CH, H), jnp.float32) for _ in range(KB)]
            + [pltpu.SemaphoreType.DMA for _ in range(KB)]
        ),
    )
    def k(a_hbm, p_hbm, src_hbm, dst_hbm, out_hbm, sidx, didx, psp, *rest):
        bufs, sems = rest[:KB], rest[KB:]
        c = lax.axis_index("c")
        s = lax.axis_index("s")
        wid = s * NC + c
        base = wid * EPW
        pltpu.sync_copy(src_hbm.at[wid], sidx)
        pltpu.sync_copy(dst_hbm.at[wid], didx)
        pltpu.sync_copy(p_hbm.at[pl.ds(s * prows, prows)],
                        psp.at[pl.ds(s * prows, prows)])
        plsc.subcore_barrier()

        @pl.loop(0, NCH // KB)
        def _(g):
            j0 = g * KB
            dA = [pltpu.async_copy(a_hbm.at[sidx.at[j0 + b]], bufs[b], sems[b])
                  for b in range(KB)]
            dP = []
            for b in range(KB):
                dA[b].wait()
                dP.append(pltpu.async_copy(psp.at[didx.at[j0 + b]], bufs[b],
                                           sems[b], add=True))
            dW = []
            for b in range(KB):
                dP[b].wait()
                dW.append(pltpu.async_copy(
                    bufs[b], out_hbm.at[pl.ds(base + (j0 + b) * CH, CH)],
                    sems[b]))
            for b in range(KB):
                dW[b].wait()

    return k(tableA, tableP, src3, dst3)


def _sc_scatter(messages, dst3, zeros_hbm):
    rows = PB // NS
    KS = 2

    @functools.partial(
        pl.kernel,
        out_type=jax.ShapeDtypeStruct((NC, PB, H), jnp.float32),
        mesh=_sc_mesh(),
        scratch_types=(
            [pltpu.VMEM((NCH, CH), jnp.int32),
             pltpu.VMEM_SHARED((PB, H), jnp.float32)]
            + [pltpu.VMEM((CH, H), jnp.float32) for _ in range(KS)]
            + [pltpu.SemaphoreType.DMA for _ in range(KS)]
        ),
    )
    def k(m_hbm, dst_hbm, z_hbm, out_hbm, didx, acc, *rest):
        bufs, sems = rest[:KS], rest[KS:]
        c = lax.axis_index("c")
        s = lax.axis_index("s")
        wid = s * NC + c
        base = wid * EPW
        pltpu.sync_copy(dst_hbm.at[wid], didx)
        pltpu.sync_copy(z_hbm.at[pl.ds(s * rows, rows)],
                        acc.at[pl.ds(s * rows, rows)])
        plsc.subcore_barrier()

        @pl.loop(0, NCH // KS)
        def _(g):
            j0 = g * KS
            dR = [pltpu.async_copy(
                      m_hbm.at[pl.ds(base + (j0 + b) * CH, CH)], bufs[b],
                      sems[b]) for b in range(KS)]
            dS = []
            for b in range(KS):
                dR[b].wait()
                dS.append(pltpu.async_copy(bufs[b], acc.at[didx.at[j0 + b]],
                                           sems[b], add=True))
            for b in range(KS):
                dS[b].wait()

        plsc.subcore_barrier()
        pltpu.sync_copy(acc.at[pl.ds(s * rows, rows)],
                        out_hbm.at[c, pl.ds(s * rows, rows)])

    return k(messages, dst3, zeros_hbm)


def kernel(atom_rep_0, atom_rep_1, atom_rep_2, probe_edges_features, params,
           probe_edges):
    f32 = jnp.float32
    src = probe_edges[:, 0].astype(jnp.int32)
    dst = probe_edges[:, 1].astype(jnp.int32)
    pad = EP - E
    src3 = jnp.concatenate([src, jnp.zeros((pad,), jnp.int32)]).reshape(NW, NCH, CH)
    dst3 = jnp.concatenate([dst, jnp.full((pad,), NP, jnp.int32)]).reshape(NW, NCH, CH)
    d_pad = jnp.concatenate(
        [probe_edges_features.astype(f32),
         jnp.full((pad, 1), 2.0 * CUTOFF, f32)], axis=0)

    waT, wbT, b1s, m2T, b2s, f1T, bf1, f2T, bf2 = [], [], [], [], [], [], [], [], []
    for i in range(NI):
        p = params["msg"][i]
        w1 = p["m1"][0]
        waT.append(w1[:, :H].T)
        wbT.append(w1[:, H:].T)
        b1s.append(p["m1"][1][None, :])
        m2T.append(p["m2"][0].T)
        b2s.append(p["m2"][1][None, :])
        f1T.append(jnp.zeros((ESP, H), f32).at[:ES].set(p["f1"][0].T))
        bf1.append(p["f1"][1][None, :])
        f2T.append(p["f2"][0].T)
        bf2.append(p["f2"][1][None, :])
    gw = [[params["gate"][i][k][0].T for k in ("g1", "g2")] for i in range(NI)]
    gb = [[params["gate"][i][k][1][None, :] for k in ("g1", "g2")] for i in range(NI)]
    sw = [[params["state"][i][k][0].T for k in ("s1", "s2")] for i in range(NI)]
    sb = [[params["state"][i][k][1][None, :] for k in ("s1", "s2")] for i in range(NI)]
    r1T = params["readout"]["r1"][0].T
    br1 = params["readout"]["r1"][1][None, :]
    r2T = jnp.zeros((H, 8), f32).at[:, 0].set(params["readout"]["r2"][0][0])
    br2 = jnp.zeros((1, 8), f32).at[0, 0].set(params["readout"]["r2"][1][0])

    mu = jnp.arange(0.0, CUTOFF, STEP, dtype=f32)
    mu_pad = jnp.full((1, ESP), 1.0e3, f32).at[0, :ES].set(mu)

    zeros_pb = jnp.zeros((PB, H), f32)
    ones_row = jnp.ones((1, H), f32)

    atoms_s = jnp.zeros((NI, NAP, H), f32).at[:, :NA].set(
        jnp.stack([atom_rep_0, atom_rep_1, atom_rep_2]))
    A_all = _precompute_A(atoms_s, jnp.stack(waT))

    cc_col = _cc_precompute(d_pad.reshape(EP // H, H)).reshape(EP, 1)

    probe = zeros_pb
    P = None
    for i in range(NI):
        if i == 0:
            hidden = _sc_gather1(A_all[0], src3)
        else:
            hidden = _sc_gather2(A_all[i], P, src3, dst3)
        messages = _edge_mlp(hidden, d_pad, cc_col, ones_row, b1s[i],
                             m2T[i], b2s[i], f1T[i], bf1[i], f2T[i], bf2[i],
                             mu_pad)
        partials = _sc_scatter(messages, dst3, zeros_pb)
        gwi = [gw[i][0], gb[i][0], gw[i][1], gb[i][1]]
        swi = [sw[i][0], sb[i][0], sw[i][1], sb[i][1]]
        if i < NI - 1:
            probe, P = _probe_step(partials, probe, gwi, swi, wbT[i + 1])
        else:
            ro = _probe_final(partials, probe, gwi, swi, [r1T, br1, r2T, br2])
    return ro[:NP, 0][None, :]


# speedup vs baseline: 2.6529x; 1.0154x over previous
import functools
import math

import jax
import jax.numpy as jnp
from jax import lax
from jax.experimental import pallas as pl
from jax.experimental.pallas import tpu as pltpu
from jax.experimental.pallas import tpu_sc as plsc

H = 128
CUTOFF = 5.0
STEP = 0.1
ES = 50
ESP = 64
NI = 3
NA = 10000
NAP = 10240
NP = 8000
E = 128000
PB = 8192
NC = 2
NS = 16
NW = NC * NS
EP = 131072
EPW = EP // NW
CH = 128
NCH = EPW // CH
LOG2 = float(math.log(2.0))
BE = 2048
BP = 1024


def _ssp(x):
    return jnp.maximum(x, 0.0) + jnp.log1p(jnp.exp(-jnp.abs(x))) - LOG2


def _pre_body(a_ref, w_ref, o_ref):
    o_ref[0] = jnp.dot(a_ref[0], w_ref[0], preferred_element_type=jnp.float32)


def _precompute_A(atoms_s, waT_s):
    return pl.pallas_call(
        _pre_body,
        grid=(NI,),
        in_specs=[
            pl.BlockSpec((1, NAP, H), lambda i: (i, 0, 0)),
            pl.BlockSpec((1, H, H), lambda i: (i, 0, 0)),
        ],
        out_specs=pl.BlockSpec((1, NAP, H), lambda i: (i, 0, 0)),
        out_shape=jax.ShapeDtypeStruct((NI, NAP, H), jnp.float32),
    )(atoms_s, waT_s)


def _cc_body(d_ref, o_ref):
    d = d_ref[...]
    o_ref[...] = jnp.where(
        d < CUTOFF, 0.5 * (jnp.cos((jnp.pi / CUTOFF) * d) + 1.0), 0.0)


def _cc_precompute(d_sq):
    return pl.pallas_call(
        _cc_body,
        out_shape=jax.ShapeDtypeStruct((EP // H, H), jnp.float32),
    )(d_sq)


def _edge_body(h_ref, d_ref, cc_ref, ones_ref, b1_ref, m2_ref, b2_ref,
               f1_ref, bf1_ref, f2_ref, bf2_ref, mu_ref, o_ref):
    ccH = jnp.dot(cc_ref[...], ones_ref[...],
                  preferred_element_type=jnp.float32)
    dE = jnp.dot(d_ref[...], ones_ref[:, :ESP],
                 preferred_element_type=jnp.float32)
    h = _ssp(h_ref[...] + b1_ref[...])
    m = jnp.dot(h, m2_ref[...], preferred_element_type=jnp.float32) + b2_ref[...]
    es = jnp.exp((-1.0 / (2.0 * STEP * STEP)) * (dE - mu_ref[...]) ** 2)
    t = _ssp(jnp.dot(es, f1_ref[...], preferred_element_type=jnp.float32)
             + bf1_ref[...])
    fw = jnp.dot(t, f2_ref[...], preferred_element_type=jnp.float32) + bf2_ref[...]
    o_ref[...] = m * (fw * ccH)


def _edge_mlp(hidden, d_pad, cc_col, ones_row, b1, m2T, b2, f1T, bf1, f2T,
              bf2, mu_pad):
    full = lambda r, c: pl.BlockSpec((r, c), lambda i: (0, 0))
    return pl.pallas_call(
        _edge_body,
        grid=(EP // BE,),
        in_specs=[
            pl.BlockSpec((BE, H), lambda i: (i, 0)),
            pl.BlockSpec((BE, 1), lambda i: (i, 0)),
            pl.BlockSpec((BE, 1), lambda i: (i, 0)),
            full(1, H),
            full(1, H), full(H, H), full(1, H),
            full(ESP, H), full(1, H), full(H, H), full(1, H), full(1, ESP),
        ],
        out_specs=pl.BlockSpec((BE, H), lambda i: (i, 0)),
        out_shape=jax.ShapeDtypeStruct((EP, H), jnp.float32),
    )(hidden, d_pad, cc_col, ones_row, b1, m2T, b2, f1T, bf1, f2T, bf2,
      mu_pad)


def _probe_common(p_ref, ps_ref, g1, bg1, g2, bg2, s1, bs1, s2, bs2):
    msgsum = p_ref[0] + p_ref[1]
    probe = ps_ref[...]
    gates = jax.nn.sigmoid(
        jnp.dot(_ssp(jnp.dot(probe, g1[...], preferred_element_type=jnp.float32)
                     + bg1[...]), g2[...], preferred_element_type=jnp.float32)
        + bg2[...])
    trans = (jnp.dot(_ssp(jnp.dot(msgsum, s1[...],
                                  preferred_element_type=jnp.float32) + bs1[...]),
                     s2[...], preferred_element_type=jnp.float32) + bs2[...])
    return probe * gates + (1.0 - gates) * trans


def _probe_step_body(p_ref, ps_ref, g1, bg1, g2, bg2, s1, bs1, s2, bs2, wbn,
                     np_ref, pn_ref):
    newp = _probe_common(p_ref, ps_ref, g1, bg1, g2, bg2, s1, bs1, s2, bs2)
    np_ref[...] = newp
    pn_ref[...] = jnp.dot(newp, wbn[...], preferred_element_type=jnp.float32)


def _probe_final_body(p_ref, ps_ref, g1, bg1, g2, bg2, s1, bs1, s2, bs2,
                      r1, br1, r2, br2, o_ref):
    newp = _probe_common(p_ref, ps_ref, g1, bg1, g2, bg2, s1, bs1, s2, bs2)
    ro = jnp.dot(_ssp(jnp.dot(newp, r1[...], preferred_element_type=jnp.float32)
                      + br1[...]), r2[...], preferred_element_type=jnp.float32)
    o_ref[...] = ro + br2[...]


def _probe_step(partials, probe, gw, sw, wbnT):
    full = lambda r, c: pl.BlockSpec((r, c), lambda i: (0, 0))
    row = pl.BlockSpec((BP, H), lambda i: (i, 0))
    return pl.pallas_call(
        _probe_step_body,
        grid=(PB // BP,),
        in_specs=[
            pl.BlockSpec((NC, BP, H), lambda i: (0, i, 0)), row,
            full(H, H), full(1, H), full(H, H), full(1, H),
            full(H, H), full(1, H), full(H, H), full(1, H),
            full(H, H),
        ],
        out_specs=[row, row],
        out_shape=[jax.ShapeDtypeStruct((PB, H), jnp.float32),
                   jax.ShapeDtypeStruct((PB, H), jnp.float32)],
    )(partials, probe, *gw, *sw, wbnT)


def _probe_final(partials, probe, gw, sw, rw):
    full = lambda r, c: pl.BlockSpec((r, c), lambda i: (0, 0))
    row = pl.BlockSpec((BP, H), lambda i: (i, 0))
    return pl.pallas_call(
        _probe_final_body,
        grid=(PB // BP,),
        in_specs=[
            pl.BlockSpec((NC, BP, H), lambda i: (0, i, 0)), row,
            full(H, H), full(1, H), full(H, H), full(1, H),
            full(H, H), full(1, H), full(H, H), full(1, H),
            full(H, H), full(1, H), full(H, 8), full(1, 8),
        ],
        out_specs=pl.BlockSpec((BP, 8), lambda i: (i, 0)),
        out_shape=jax.ShapeDtypeStruct((PB, 8), jnp.float32),
    )(partials, probe, *gw, *sw, *rw)


def _sc_mesh():
    return plsc.VectorSubcoreMesh(core_axis_name="c", subcore_axis_name="s",
                                  num_cores=NC)


def _sc_gather1(table, src3):
    KB = 2
    trows = NAP // NS

    @functools.partial(
        pl.kernel,
        out_type=jax.ShapeDtypeStruct((EP, H), jnp.float32),
        mesh=_sc_mesh(),
        scratch_types=(
            [pltpu.VMEM((NCH, CH), jnp.int32),
             pltpu.VMEM_SHARED((NAP, H), jnp.float32)]
            + [pltpu.VMEM((CH, H), jnp.float32) for _ in range(KB)]
            + [pltpu.SemaphoreType.DMA for _ in range(KB)]
        ),
    )
    def k(t_hbm, src_hbm, out_hbm, sidx, tsp, *rest):
        bufs, sems = rest[:KB], rest[KB:]
        c = lax.axis_index("c")
        s = lax.axis_index("s")
        wid = s * NC + c
        base = wid * EPW
        pltpu.sync_copy(src_hbm.at[wid], sidx)
        pltpu.sync_copy(t_hbm.at[pl.ds(s * trows, trows)],
                        tsp.at[pl.ds(s * trows, trows)])
        plsc.subcore_barrier()

        @pl.loop(0, NCH // KB)
        def _(g):
            j0 = g * KB
            dA = [pltpu.async_copy(tsp.at[sidx.at[j0 + b]], bufs[b], sems[b])
                  for b in range(KB)]
            dW = []
            for b in range(KB):
                dA[b].wait()
                dW.append(pltpu.async_copy(
                    bufs[b], out_hbm.at[pl.ds(base + (j0 + b) * CH, CH)],
                    sems[b]))
            for b in range(KB):
                dW[b].wait()

    return k(table, src3)


CH2 = 64
NCH2 = EPW // CH2


def _sc_gather2(tableA, tableP, src3, dst3):
    KB = 4
    prows = PB // NS

    @functools.partial(
        pl.kernel,
        out_type=jax.ShapeDtypeStruct((EP, H), jnp.float32),
        mesh=_sc_mesh(),
        scratch_types=(
            [pltpu.VMEM((NCH2, CH2), jnp.int32),
             pltpu.VMEM((NCH2, CH2), jnp.int32),
             pltpu.VMEM_SHARED((PB, H), jnp.float32)]
            + [pltpu.VMEM((CH2, H), jnp.float32) for _ in range(KB)]
            + [pltpu.SemaphoreType.DMA for _ in range(KB)]
        ),
    )
    def k(a_hbm, p_hbm, src_hbm, dst_hbm, out_hbm, sidx, didx, psp, *rest):
        bufs, sems = rest[:KB], rest[KB:]
        c = lax.axis_index("c")
        s = lax.axis_index("s")
        wid = s * NC + c
        base = wid * EPW
        pltpu.sync_copy(src_hbm.at[wid], sidx)
        pltpu.sync_copy(dst_hbm.at[wid], didx)
        pltpu.sync_copy(p_hbm.at[pl.ds(s * prows, prows)],
                        psp.at[pl.ds(s * prows, prows)])
        plsc.subcore_barrier()

        @pl.loop(0, NCH2 // KB)
        def _(g):
            j0 = g * KB
            dA = [pltpu.async_copy(a_hbm.at[sidx.at[j0 + b]], bufs[b], sems[b])
                  for b in range(KB)]
            dP = []
            for b in range(KB):
                dA[b].wait()
                dP.append(pltpu.async_copy(psp.at[didx.at[j0 + b]], bufs[b],
                                           sems[b], add=True))
            dW = []
            for b in range(KB):
                dP[b].wait()
                dW.append(pltpu.async_copy(
                    bufs[b], out_hbm.at[pl.ds(base + (j0 + b) * CH2, CH2)],
                    sems[b]))
            for b in range(KB):
                dW[b].wait()

    return k(tableA, tableP, src3, dst3)


def _sc_scatter(messages, dst3, zeros_hbm):
    rows = PB // NS
    KS = 2

    @functools.partial(
        pl.kernel,
        out_type=jax.ShapeDtypeStruct((NC, PB, H), jnp.float32),
        mesh=_sc_mesh(),
        scratch_types=(
            [pltpu.VMEM((NCH, CH), jnp.int32),
             pltpu.VMEM_SHARED((PB, H), jnp.float32)]
            + [pltpu.VMEM((CH, H), jnp.float32) for _ in range(KS)]
            + [pltpu.SemaphoreType.DMA for _ in range(KS)]
        ),
    )
    def k(m_hbm, dst_hbm, z_hbm, out_hbm, didx, acc, *rest):
        bufs, sems = rest[:KS], rest[KS:]
        c = lax.axis_index("c")
        s = lax.axis_index("s")
        wid = s * NC + c
        base = wid * EPW
        pltpu.sync_copy(dst_hbm.at[wid], didx)
        pltpu.sync_copy(z_hbm.at[pl.ds(s * rows, rows)],
                        acc.at[pl.ds(s * rows, rows)])
        plsc.subcore_barrier()

        @pl.loop(0, NCH // KS)
        def _(g):
            j0 = g * KS
            dR = [pltpu.async_copy(
                      m_hbm.at[pl.ds(base + (j0 + b) * CH, CH)], bufs[b],
                      sems[b]) for b in range(KS)]
            dS = []
            for b in range(KS):
                dR[b].wait()
                dS.append(pltpu.async_copy(bufs[b], acc.at[didx.at[j0 + b]],
                                           sems[b], add=True))
            for b in range(KS):
                dS[b].wait()

        plsc.subcore_barrier()
        pltpu.sync_copy(acc.at[pl.ds(s * rows, rows)],
                        out_hbm.at[c, pl.ds(s * rows, rows)])

    return k(messages, dst3, zeros_hbm)


def kernel(atom_rep_0, atom_rep_1, atom_rep_2, probe_edges_features, params,
           probe_edges):
    f32 = jnp.float32
    src = probe_edges[:, 0].astype(jnp.int32)
    dst = probe_edges[:, 1].astype(jnp.int32)
    pad = EP - E
    src_p = jnp.concatenate([src, jnp.zeros((pad,), jnp.int32)])
    dst_p = jnp.concatenate([dst, jnp.full((pad,), NP, jnp.int32)])
    src3 = src_p.reshape(NW, NCH, CH)
    dst3 = dst_p.reshape(NW, NCH, CH)
    src3b = src_p.reshape(NW, NCH2, CH2)
    dst3b = dst_p.reshape(NW, NCH2, CH2)
    d_pad = jnp.concatenate(
        [probe_edges_features.astype(f32),
         jnp.full((pad, 1), 2.0 * CUTOFF, f32)], axis=0)

    waT, wbT, b1s, m2T, b2s, f1T, bf1, f2T, bf2 = [], [], [], [], [], [], [], [], []
    for i in range(NI):
        p = params["msg"][i]
        w1 = p["m1"][0]
        waT.append(w1[:, :H].T)
        wbT.append(w1[:, H:].T)
        b1s.append(p["m1"][1][None, :])
        m2T.append(p["m2"][0].T)
        b2s.append(p["m2"][1][None, :])
        f1T.append(jnp.zeros((ESP, H), f32).at[:ES].set(p["f1"][0].T))
        bf1.append(p["f1"][1][None, :])
        f2T.append(p["f2"][0].T)
        bf2.append(p["f2"][1][None, :])
    gw = [[params["gate"][i][k][0].T for k in ("g1", "g2")] for i in range(NI)]
    gb = [[params["gate"][i][k][1][None, :] for k in ("g1", "g2")] for i in range(NI)]
    sw = [[params["state"][i][k][0].T for k in ("s1", "s2")] for i in range(NI)]
    sb = [[params["state"][i][k][1][None, :] for k in ("s1", "s2")] for i in range(NI)]
    r1T = params["readout"]["r1"][0].T
    br1 = params["readout"]["r1"][1][None, :]
    r2T = jnp.zeros((H, 8), f32).at[:, 0].set(params["readout"]["r2"][0][0])
    br2 = jnp.zeros((1, 8), f32).at[0, 0].set(params["readout"]["r2"][1][0])

    mu = jnp.arange(0.0, CUTOFF, STEP, dtype=f32)
    mu_pad = jnp.full((1, ESP), 1.0e3, f32).at[0, :ES].set(mu)

    zeros_pb = jnp.zeros((PB, H), f32)
    ones_row = jnp.ones((1, H), f32)

    atoms_s = jnp.zeros((NI, NAP, H), f32).at[:, :NA].set(
        jnp.stack([atom_rep_0, atom_rep_1, atom_rep_2]))
    A_all = _precompute_A(atoms_s, jnp.stack(waT))

    cc_col = _cc_precompute(d_pad.reshape(EP // H, H)).reshape(EP, 1)

    probe = zeros_pb
    P = None
    for i in range(NI):
        if i == 0:
            hidden = _sc_gather1(A_all[0], src3)
        else:
            hidden = _sc_gather2(A_all[i], P, src3b, dst3b)
        messages = _edge_mlp(hidden, d_pad, cc_col, ones_row, b1s[i],
                             m2T[i], b2s[i], f1T[i], bf1[i], f2T[i], bf2[i],
                             mu_pad)
        partials = _sc_scatter(messages, dst3, zeros_pb)
        gwi = [gw[i][0], gb[i][0], gw[i][1], gb[i][1]]
        swi = [sw[i][0], sb[i][0], sw[i][1], sb[i][1]]
        if i < NI - 1:
            probe, P = _probe_step(partials, probe, gwi, swi, wbT[i + 1])
        else:
            ro = _probe_final(partials, probe, gwi, swi, [r1T, br1, r2T, br2])
    return ro[:NP, 0][None, :]


# speedup vs baseline: 3.0785x; 1.1604x over previous
import functools
import math

import jax
import jax.numpy as jnp
from jax import lax
from jax.experimental import pallas as pl
from jax.experimental.pallas import tpu as pltpu
from jax.experimental.pallas import tpu_sc as plsc

H = 128
CUTOFF = 5.0
STEP = 0.1
ES = 50
ESP = 64
NI = 3
NA = 10000
NAP = 10240
NP = 8000
E = 128000
PB = 8192
NC = 2
NS = 16
NW = NC * NS
EP = 131072
EPW = EP // NW
CH = 128
NCH = EPW // CH
LOG2 = float(math.log(2.0))
BE = 2048
BP = 1024


def _ssp(x):
    return jnp.maximum(x, 0.0) + jnp.log1p(jnp.exp(-jnp.abs(x))) - LOG2


def _pre_body(a_ref, w_ref, o_ref):
    r = jnp.dot(a_ref[0], w_ref[0], preferred_element_type=jnp.float32)
    o_ref[0, 0] = r[:, :HH]
    o_ref[0, 1] = r[:, HH:]


def _precompute_A(atoms_s, waT_s):
    return pl.pallas_call(
        _pre_body,
        grid=(NI,),
        in_specs=[
            pl.BlockSpec((1, NAP, H), lambda i: (i, 0, 0)),
            pl.BlockSpec((1, H, H), lambda i: (i, 0, 0)),
        ],
        out_specs=pl.BlockSpec((1, NC, NAP, HH), lambda i: (i, 0, 0, 0)),
        out_shape=jax.ShapeDtypeStruct((NI, NC, NAP, HH), jnp.float32),
    )(atoms_s, waT_s)


def _cc_body(d_ref, o_ref):
    d = d_ref[...]
    o_ref[...] = jnp.where(
        d < CUTOFF, 0.5 * (jnp.cos((jnp.pi / CUTOFF) * d) + 1.0), 0.0)


def _cc_precompute(d_sq):
    return pl.pallas_call(
        _cc_body,
        out_shape=jax.ShapeDtypeStruct((EP // H, H), jnp.float32),
    )(d_sq)


def _edge_body(h_ref, d_ref, cc_ref, ones_ref, b1l_ref, b1r_ref, m2a_ref,
               m2b_ref, b2_ref, f1_ref, bf1_ref, f2_ref, bf2_ref, mu_ref,
               o_ref):
    ccH = jnp.dot(cc_ref[...], ones_ref[...],
                  preferred_element_type=jnp.float32)
    dE = jnp.dot(d_ref[...], ones_ref[:, :ESP],
                 preferred_element_type=jnp.float32)
    h0 = _ssp(h_ref[0] + b1l_ref[...])
    h1 = _ssp(h_ref[1] + b1r_ref[...])
    m = (jnp.dot(h0, m2a_ref[...], preferred_element_type=jnp.float32)
         + jnp.dot(h1, m2b_ref[...], preferred_element_type=jnp.float32)
         + b2_ref[...])
    es = jnp.exp((-1.0 / (2.0 * STEP * STEP)) * (dE - mu_ref[...]) ** 2)
    t = _ssp(jnp.dot(es, f1_ref[...], preferred_element_type=jnp.float32)
             + bf1_ref[...])
    fw = jnp.dot(t, f2_ref[...], preferred_element_type=jnp.float32) + bf2_ref[...]
    o_ref[...] = m * (fw * ccH)


def _edge_mlp(hidden, d_pad, cc_col, ones_row, b1l, b1r, m2a, m2b, b2, f1T,
              bf1, f2T, bf2, mu_pad):
    full = lambda r, c: pl.BlockSpec((r, c), lambda i: (0, 0))
    return pl.pallas_call(
        _edge_body,
        grid=(EP // BE,),
        in_specs=[
            pl.BlockSpec((NC, BE, HH), lambda i: (0, i, 0)),
            pl.BlockSpec((BE, 1), lambda i: (i, 0)),
            pl.BlockSpec((BE, 1), lambda i: (i, 0)),
            full(1, H),
            full(1, HH), full(1, HH), full(HH, H), full(HH, H), full(1, H),
            full(ESP, H), full(1, H), full(H, H), full(1, H), full(1, ESP),
        ],
        out_specs=pl.BlockSpec((BE, H), lambda i: (i, 0)),
        out_shape=jax.ShapeDtypeStruct((EP, H), jnp.float32),
    )(hidden, d_pad, cc_col, ones_row, b1l, b1r, m2a, m2b, b2, f1T, bf1,
      f2T, bf2, mu_pad)


def _probe_common(p_ref, ps_ref, g1, bg1, g2, bg2, s1, bs1, s2, bs2):
    msgsum = p_ref[0] + p_ref[1]
    probe = ps_ref[...]
    gates = jax.nn.sigmoid(
        jnp.dot(_ssp(jnp.dot(probe, g1[...], preferred_element_type=jnp.float32)
                     + bg1[...]), g2[...], preferred_element_type=jnp.float32)
        + bg2[...])
    trans = (jnp.dot(_ssp(jnp.dot(msgsum, s1[...],
                                  preferred_element_type=jnp.float32) + bs1[...]),
                     s2[...], preferred_element_type=jnp.float32) + bs2[...])
    return probe * gates + (1.0 - gates) * trans


def _probe_step_body(p_ref, ps_ref, g1, bg1, g2, bg2, s1, bs1, s2, bs2, wbn,
                     np_ref, pn_ref):
    newp = _probe_common(p_ref, ps_ref, g1, bg1, g2, bg2, s1, bs1, s2, bs2)
    np_ref[...] = newp
    pn = jnp.dot(newp, wbn[...], preferred_element_type=jnp.float32)
    pn_ref[0] = pn[:, :HH]
    pn_ref[1] = pn[:, HH:]


def _probe_final_body(p_ref, ps_ref, g1, bg1, g2, bg2, s1, bs1, s2, bs2,
                      r1, br1, r2, br2, o_ref):
    newp = _probe_common(p_ref, ps_ref, g1, bg1, g2, bg2, s1, bs1, s2, bs2)
    ro = jnp.dot(_ssp(jnp.dot(newp, r1[...], preferred_element_type=jnp.float32)
                      + br1[...]), r2[...], preferred_element_type=jnp.float32)
    o_ref[...] = ro + br2[...]


def _probe_step(partials, probe, gw, sw, wbnT):
    full = lambda r, c: pl.BlockSpec((r, c), lambda i: (0, 0))
    row = pl.BlockSpec((BP, H), lambda i: (i, 0))
    return pl.pallas_call(
        _probe_step_body,
        grid=(PB // BP,),
        in_specs=[
            pl.BlockSpec((NC, BP, H), lambda i: (0, i, 0)), row,
            full(H, H), full(1, H), full(H, H), full(1, H),
            full(H, H), full(1, H), full(H, H), full(1, H),
            full(H, H),
        ],
        out_specs=[row, pl.BlockSpec((NC, BP, HH), lambda i: (0, i, 0))],
        out_shape=[jax.ShapeDtypeStruct((PB, H), jnp.float32),
                   jax.ShapeDtypeStruct((NC, PB, HH), jnp.float32)],
    )(partials, probe, *gw, *sw, wbnT)


def _probe_final(partials, probe, gw, sw, rw):
    full = lambda r, c: pl.BlockSpec((r, c), lambda i: (0, 0))
    row = pl.BlockSpec((BP, H), lambda i: (i, 0))
    return pl.pallas_call(
        _probe_final_body,
        grid=(PB // BP,),
        in_specs=[
            pl.BlockSpec((NC, BP, H), lambda i: (0, i, 0)), row,
            full(H, H), full(1, H), full(H, H), full(1, H),
            full(H, H), full(1, H), full(H, H), full(1, H),
            full(H, H), full(1, H), full(H, 8), full(1, 8),
        ],
        out_specs=pl.BlockSpec((BP, 8), lambda i: (i, 0)),
        out_shape=jax.ShapeDtypeStruct((PB, 8), jnp.float32),
    )(partials, probe, *gw, *sw, *rw)


def _sc_mesh():
    return plsc.VectorSubcoreMesh(core_axis_name="c", subcore_axis_name="s",
                                  num_cores=NC)


def _sc_gather1(tableA, srcT):
    KB = 4
    arows = NAP // NS

    @functools.partial(
        pl.kernel,
        out_type=jax.ShapeDtypeStruct((NC, EP, HH), jnp.float32),
        mesh=_sc_mesh(),
        scratch_types=(
            [pltpu.VMEM((NCHT, CHT), jnp.int32),
             pltpu.VMEM_SHARED((NAP, HH), jnp.float32)]
            + [pltpu.VMEM((CHT, HH), jnp.float32) for _ in range(KB)]
            + [pltpu.SemaphoreType.DMA for _ in range(KB)]
        ),
    )
    def k(a_hbm, src_hbm, out_hbm, sidx, asp, *rest):
        bufs, sems = rest[:KB], rest[KB:]
        c = lax.axis_index("c")
        s = lax.axis_index("s")
        base = s * EPT
        pltpu.sync_copy(src_hbm.at[s], sidx)
        pltpu.sync_copy(a_hbm.at[c, pl.ds(s * arows, arows)],
                        asp.at[pl.ds(s * arows, arows)])
        plsc.subcore_barrier()

        @pl.loop(0, NCHT // KB)
        def _(g):
            j0 = g * KB
            dA = [pltpu.async_copy(asp.at[sidx.at[j0 + b]], bufs[b], sems[b])
                  for b in range(KB)]
            dW = []
            for b in range(KB):
                dA[b].wait()
                dW.append(pltpu.async_copy(
                    bufs[b],
                    out_hbm.at[c, pl.ds(base + (j0 + b) * CHT, CHT)],
                    sems[b]))
            for b in range(KB):
                dW[b].wait()

    return k(tableA, srcT)


HH = H // 2
EPT = EP // NS
CHT = 128
NCHT = EPT // CHT


def _sc_gather2(tableA, tableP, srcT, dstT):
    KB = 2
    arows = NAP // NS
    prows = PB // NS

    @functools.partial(
        pl.kernel,
        out_type=jax.ShapeDtypeStruct((NC, EP, HH), jnp.float32),
        mesh=_sc_mesh(),
        scratch_types=(
            [pltpu.VMEM((NCHT, CHT), jnp.int32),
             pltpu.VMEM((NCHT, CHT), jnp.int32),
             pltpu.VMEM_SHARED((NAP, HH), jnp.float32),
             pltpu.VMEM_SHARED((PB, HH), jnp.float32)]
            + [pltpu.VMEM((CHT, HH), jnp.float32) for _ in range(KB)]
            + [pltpu.SemaphoreType.DMA for _ in range(KB)]
        ),
    )
    def k(a_hbm, p_hbm, src_hbm, dst_hbm, out_hbm, sidx, didx, asp, psp,
          *rest):
        bufs, sems = rest[:KB], rest[KB:]
        c = lax.axis_index("c")
        s = lax.axis_index("s")
        base = s * EPT
        pltpu.sync_copy(src_hbm.at[s], sidx)
        pltpu.sync_copy(dst_hbm.at[s], didx)
        pltpu.sync_copy(a_hbm.at[c, pl.ds(s * arows, arows)],
                        asp.at[pl.ds(s * arows, arows)])
        pltpu.sync_copy(p_hbm.at[c, pl.ds(s * prows, prows)],
                        psp.at[pl.ds(s * prows, prows)])
        plsc.subcore_barrier()

        @pl.loop(0, NCHT // KB)
        def _(g):
            j0 = g * KB
            dA = [pltpu.async_copy(asp.at[sidx.at[j0 + b]], bufs[b], sems[b])
                  for b in range(KB)]
            dP = []
            for b in range(KB):
                dA[b].wait()
                dP.append(pltpu.async_copy(psp.at[didx.at[j0 + b]], bufs[b],
                                           sems[b], add=True))
            dW = []
            for b in range(KB):
                dP[b].wait()
                dW.append(pltpu.async_copy(
                    bufs[b],
                    out_hbm.at[c, pl.ds(base + (j0 + b) * CHT, CHT)],
                    sems[b]))
            for b in range(KB):
                dW[b].wait()

    return k(tableA, tableP, srcT, dstT)


def _sc_scatter(messages, dst3, zeros_hbm):
    rows = PB // NS
    KS = 2

    @functools.partial(
        pl.kernel,
        out_type=jax.ShapeDtypeStruct((NC, PB, H), jnp.float32),
        mesh=_sc_mesh(),
        scratch_types=(
            [pltpu.VMEM((NCH, CH), jnp.int32),
             pltpu.VMEM_SHARED((PB, H), jnp.float32)]
            + [pltpu.VMEM((CH, H), jnp.float32) for _ in range(KS)]
            + [pltpu.SemaphoreType.DMA for _ in range(KS)]
        ),
    )
    def k(m_hbm, dst_hbm, z_hbm, out_hbm, didx, acc, *rest):
        bufs, sems = rest[:KS], rest[KS:]
        c = lax.axis_index("c")
        s = lax.axis_index("s")
        wid = s * NC + c
        base = wid * EPW
        pltpu.sync_copy(dst_hbm.at[wid], didx)
        pltpu.sync_copy(z_hbm.at[pl.ds(s * rows, rows)],
                        acc.at[pl.ds(s * rows, rows)])
        plsc.subcore_barrier()

        @pl.loop(0, NCH // KS)
        def _(g):
            j0 = g * KS
            dR = [pltpu.async_copy(
                      m_hbm.at[pl.ds(base + (j0 + b) * CH, CH)], bufs[b],
                      sems[b]) for b in range(KS)]
            dS = []
            for b in range(KS):
                dR[b].wait()
                dS.append(pltpu.async_copy(bufs[b], acc.at[didx.at[j0 + b]],
                                           sems[b], add=True))
            for b in range(KS):
                dS[b].wait()

        plsc.subcore_barrier()
        pltpu.sync_copy(acc.at[pl.ds(s * rows, rows)],
                        out_hbm.at[c, pl.ds(s * rows, rows)])

    return k(messages, dst3, zeros_hbm)


def kernel(atom_rep_0, atom_rep_1, atom_rep_2, probe_edges_features, params,
           probe_edges):
    f32 = jnp.float32
    src = probe_edges[:, 0].astype(jnp.int32)
    dst = probe_edges[:, 1].astype(jnp.int32)
    pad = EP - E
    src_p = jnp.concatenate([src, jnp.zeros((pad,), jnp.int32)])
    dst_p = jnp.concatenate([dst, jnp.full((pad,), NP, jnp.int32)])
    src3 = src_p.reshape(NW, NCH, CH)
    dst3 = dst_p.reshape(NW, NCH, CH)
    srcT = src_p.reshape(NS, NCHT, CHT)
    dstT = dst_p.reshape(NS, NCHT, CHT)
    d_pad = jnp.concatenate(
        [probe_edges_features.astype(f32),
         jnp.full((pad, 1), 2.0 * CUTOFF, f32)], axis=0)

    waT, wbT, b1s, m2T, b2s, f1T, bf1, f2T, bf2 = [], [], [], [], [], [], [], [], []
    for i in range(NI):
        p = params["msg"][i]
        w1 = p["m1"][0]
        waT.append(w1[:, :H].T)
        wbT.append(w1[:, H:].T)
        b1s.append(p["m1"][1][None, :])
        m2T.append(p["m2"][0].T)
        b2s.append(p["m2"][1][None, :])
        f1T.append(jnp.zeros((ESP, H), f32).at[:ES].set(p["f1"][0].T))
        bf1.append(p["f1"][1][None, :])
        f2T.append(p["f2"][0].T)
        bf2.append(p["f2"][1][None, :])
    gw = [[params["gate"][i][k][0].T for k in ("g1", "g2")] for i in range(NI)]
    gb = [[params["gate"][i][k][1][None, :] for k in ("g1", "g2")] for i in range(NI)]
    sw = [[params["state"][i][k][0].T for k in ("s1", "s2")] for i in range(NI)]
    sb = [[params["state"][i][k][1][None, :] for k in ("s1", "s2")] for i in range(NI)]
    r1T = params["readout"]["r1"][0].T
    br1 = params["readout"]["r1"][1][None, :]
    r2T = jnp.zeros((H, 8), f32).at[:, 0].set(params["readout"]["r2"][0][0])
    br2 = jnp.zeros((1, 8), f32).at[0, 0].set(params["readout"]["r2"][1][0])

    mu = jnp.arange(0.0, CUTOFF, STEP, dtype=f32)
    mu_pad = jnp.full((1, ESP), 1.0e3, f32).at[0, :ES].set(mu)

    zeros_pb = jnp.zeros((PB, H), f32)
    ones_row = jnp.ones((1, H), f32)

    atoms_s = jnp.zeros((NI, NAP, H), f32).at[:, :NA].set(
        jnp.stack([atom_rep_0, atom_rep_1, atom_rep_2]))
    A_all = _precompute_A(atoms_s, jnp.stack(waT))

    cc_col = _cc_precompute(d_pad.reshape(EP // H, H)).reshape(EP, 1)

    probe = zeros_pb
    P = None
    for i in range(NI):
        if i == 0:
            hidden = _sc_gather1(A_all[0], srcT)
        else:
            hidden = _sc_gather2(A_all[i], P, srcT, dstT)
        messages = _edge_mlp(hidden, d_pad, cc_col, ones_row,
                             b1s[i][:, :HH], b1s[i][:, HH:],
                             m2T[i][:HH], m2T[i][HH:], b2s[i], f1T[i],
                             bf1[i], f2T[i], bf2[i], mu_pad)
        partials = _sc_scatter(messages, dst3, zeros_pb)
        gwi = [gw[i][0], gb[i][0], gw[i][1], gb[i][1]]
        swi = [sw[i][0], sb[i][0], sw[i][1], sb[i][1]]
        if i < NI - 1:
            probe, P = _probe_step(partials, probe, gwi, swi, wbT[i + 1])
        else:
            ro = _probe_final(partials, probe, gwi, swi, [r1T, br1, r2T, br2])
    return ro[:NP, 0][None, :]


# speedup vs baseline: 3.4248x; 1.1125x over previous
import functools
import math

import jax
import jax.numpy as jnp
from jax import lax
from jax.experimental import pallas as pl
from jax.experimental.pallas import tpu as pltpu
from jax.experimental.pallas import tpu_sc as plsc

H = 128
CUTOFF = 5.0
STEP = 0.1
ES = 50
ESP = 64
NI = 3
NA = 10000
NAP = 10240
NP = 8000
E = 128000
PB = 8192
NC = 2
NS = 16
NW = NC * NS
EP = 131072
EPW = EP // NW
CH = 128
NCH = EPW // CH
EPT = EP // NS
CHT = 128
NCHT = EPT // CHT
LOG2 = float(math.log(2.0))
BE = 2048
BP = 1024


def _ssp(x):
    return jnp.maximum(x, 0.0) + jnp.log1p(jnp.exp(-jnp.abs(x))) - LOG2


def _pre_body(a_ref, w_ref, o_ref):
    o_ref[0] = jnp.dot(a_ref[0], w_ref[0], preferred_element_type=jnp.float32)


def _precompute_A(atoms_s, waT_s):
    return pl.pallas_call(
        _pre_body,
        grid=(NI,),
        in_specs=[
            pl.BlockSpec((1, NAP, H), lambda i: (i, 0, 0)),
            pl.BlockSpec((1, H, H), lambda i: (i, 0, 0)),
        ],
        out_specs=pl.BlockSpec((1, NAP, H), lambda i: (i, 0, 0)),
        out_shape=jax.ShapeDtypeStruct((NI, NAP, H), jnp.float32),
    )(atoms_s, waT_s)


def _cc_body(d_ref, o_ref):
    d = d_ref[...]
    o_ref[...] = jnp.where(
        d < CUTOFF, 0.5 * (jnp.cos((jnp.pi / CUTOFF) * d) + 1.0), 0.0)


def _cc_precompute(d_sq):
    return pl.pallas_call(
        _cc_body,
        out_shape=jax.ShapeDtypeStruct((EP // H, H), jnp.float32),
    )(d_sq)


def _edge_core(h, d_col, cc_col, ones, b1, m2, b2, f1, bf1, f2, bf2, mu):
    ccH = jnp.dot(cc_col, ones, preferred_element_type=jnp.float32)
    dE = jnp.dot(d_col, ones[:, :ESP], preferred_element_type=jnp.float32)
    hs = _ssp(h + b1)
    m = jnp.dot(hs, m2, preferred_element_type=jnp.float32) + b2
    es = jnp.exp((-1.0 / (2.0 * STEP * STEP)) * (dE - mu) ** 2)
    t = _ssp(jnp.dot(es, f1, preferred_element_type=jnp.float32) + bf1)
    fw = jnp.dot(t, f2, preferred_element_type=jnp.float32) + bf2
    return m * (fw * ccH)


def _edge_body1(h_ref, d_ref, cc_ref, ones_ref, b1_ref, m2_ref, b2_ref,
                f1_ref, bf1_ref, f2_ref, bf2_ref, mu_ref, o_ref):
    o_ref[...] = _edge_core(
        h_ref[...], d_ref[...], cc_ref[...], ones_ref[...], b1_ref[...],
        m2_ref[...], b2_ref[...], f1_ref[...], bf1_ref[...], f2_ref[...],
        bf2_ref[...], mu_ref[...])


def _edge_body2(h2_ref, d_ref, cc_ref, ones_ref, b1_ref, m2_ref,
                b2_ref, f1_ref, bf1_ref, f2_ref, bf2_ref, mu_ref, o_ref):
    o_ref[...] = _edge_core(
        h2_ref[0] + h2_ref[1], d_ref[...], cc_ref[...], ones_ref[...],
        b1_ref[...], m2_ref[...], b2_ref[...], f1_ref[...], bf1_ref[...],
        f2_ref[...], bf2_ref[...], mu_ref[...])


def _edge_mlp(h, d_pad, cc_col, ones_row, b1, m2T, b2, f1T, bf1, f2T, bf2,
              mu_pad):
    full = lambda r, c: pl.BlockSpec((r, c), lambda i: (0, 0))
    row = pl.BlockSpec((BE, H), lambda i: (i, 0))
    col = pl.BlockSpec((BE, 1), lambda i: (i, 0))
    stacked = h.ndim == 3
    hspec = (pl.BlockSpec((NC, BE, H), lambda i: (0, i, 0)) if stacked
             else row)
    return pl.pallas_call(
        _edge_body2 if stacked else _edge_body1,
        grid=(EP // BE,),
        in_specs=[
            hspec, col, col, full(1, H),
            full(1, H), full(H, H), full(1, H),
            full(ESP, H), full(1, H), full(H, H), full(1, H), full(1, ESP),
        ],
        out_specs=row,
        out_shape=jax.ShapeDtypeStruct((EP, H), jnp.float32),
    )(h, d_pad, cc_col, ones_row, b1, m2T, b2, f1T, bf1, f2T, bf2, mu_pad)


def _probe_common(p_ref, ps_ref, g1, bg1, g2, bg2, s1, bs1, s2, bs2):
    msgsum = p_ref[0] + p_ref[1]
    probe = ps_ref[...]
    gates = jax.nn.sigmoid(
        jnp.dot(_ssp(jnp.dot(probe, g1[...], preferred_element_type=jnp.float32)
                     + bg1[...]), g2[...], preferred_element_type=jnp.float32)
        + bg2[...])
    trans = (jnp.dot(_ssp(jnp.dot(msgsum, s1[...],
                                  preferred_element_type=jnp.float32) + bs1[...]),
                     s2[...], preferred_element_type=jnp.float32) + bs2[...])
    return probe * gates + (1.0 - gates) * trans


def _probe_step_body(p_ref, ps_ref, g1, bg1, g2, bg2, s1, bs1, s2, bs2, wbn,
                     np_ref, pn_ref):
    newp = _probe_common(p_ref, ps_ref, g1, bg1, g2, bg2, s1, bs1, s2, bs2)
    np_ref[...] = newp
    pn_ref[...] = jnp.dot(newp, wbn[...], preferred_element_type=jnp.float32)


def _probe_final_body(p_ref, ps_ref, g1, bg1, g2, bg2, s1, bs1, s2, bs2,
                      r1, br1, r2, br2, o_ref):
    newp = _probe_common(p_ref, ps_ref, g1, bg1, g2, bg2, s1, bs1, s2, bs2)
    ro = jnp.dot(_ssp(jnp.dot(newp, r1[...], preferred_element_type=jnp.float32)
                      + br1[...]), r2[...], preferred_element_type=jnp.float32)
    o_ref[...] = ro + br2[...]


def _probe_step(partials, probe, gw, sw, wbnT):
    full = lambda r, c: pl.BlockSpec((r, c), lambda i: (0, 0))
    row = pl.BlockSpec((BP, H), lambda i: (i, 0))
    return pl.pallas_call(
        _probe_step_body,
        grid=(PB // BP,),
        in_specs=[
            pl.BlockSpec((NC, BP, H), lambda i: (0, i, 0)), row,
            full(H, H), full(1, H), full(H, H), full(1, H),
            full(H, H), full(1, H), full(H, H), full(1, H),
            full(H, H),
        ],
        out_specs=[row, row],
        out_shape=[jax.ShapeDtypeStruct((PB, H), jnp.float32),
                   jax.ShapeDtypeStruct((PB, H), jnp.float32)],
    )(partials, probe, *gw, *sw, wbnT)


def _probe_final(partials, probe, gw, sw, rw):
    full = lambda r, c: pl.BlockSpec((r, c), lambda i: (0, 0))
    row = pl.BlockSpec((BP, H), lambda i: (i, 0))
    return pl.pallas_call(
        _probe_final_body,
        grid=(PB // BP,),
        in_specs=[
            pl.BlockSpec((NC, BP, H), lambda i: (0, i, 0)), row,
            full(H, H), full(1, H), full(H, H), full(1, H),
            full(H, H), full(1, H), full(H, H), full(1, H),
            full(H, H), full(1, H), full(H, 8), full(1, 8),
        ],
        out_specs=pl.BlockSpec((BP, 8), lambda i: (i, 0)),
        out_shape=jax.ShapeDtypeStruct((PB, 8), jnp.float32),
    )(partials, probe, *gw, *sw, *rw)


def _sc_mesh():
    return plsc.VectorSubcoreMesh(core_axis_name="c", subcore_axis_name="s",
                                  num_cores=NC)


def _sc_gather1(table, src3):
    KB = 2
    trows = NAP // NS

    @functools.partial(
        pl.kernel,
        out_type=jax.ShapeDtypeStruct((EP, H), jnp.float32),
        mesh=_sc_mesh(),
        scratch_types=(
            [pltpu.VMEM((NCH, CH), jnp.int32),
             pltpu.VMEM_SHARED((NAP, H), jnp.float32)]
            + [pltpu.VMEM((CH, H), jnp.float32) for _ in range(KB)]
            + [pltpu.SemaphoreType.DMA for _ in range(KB)]
        ),
    )
    def k(t_hbm, src_hbm, out_hbm, sidx, tsp, *rest):
        bufs, sems = rest[:KB], rest[KB:]
        c = lax.axis_index("c")
        s = lax.axis_index("s")
        wid = s * NC + c
        base = wid * EPW
        pltpu.sync_copy(src_hbm.at[wid], sidx)
        pltpu.sync_copy(t_hbm.at[pl.ds(s * trows, trows)],
                        tsp.at[pl.ds(s * trows, trows)])
        plsc.subcore_barrier()

        @pl.loop(0, NCH // KB)
        def _(g):
            j0 = g * KB
            dA = [pltpu.async_copy(tsp.at[sidx.at[j0 + b]], bufs[b], sems[b])
                  for b in range(KB)]
            dW = []
            for b in range(KB):
                dA[b].wait()
                dW.append(pltpu.async_copy(
                    bufs[b], out_hbm.at[pl.ds(base + (j0 + b) * CH, CH)],
                    sems[b]))
            for b in range(KB):
                dW[b].wait()

    return k(table, src3)


def _sc_gather2(tableA, tableP, srcT, dstT):
    KB = 2
    trows = NAP // NS
    prows = PB // NS

    @functools.partial(
        pl.kernel,
        out_type=jax.ShapeDtypeStruct((NC, EP, H), jnp.float32),
        mesh=_sc_mesh(),
        scratch_types=(
            [pltpu.VMEM((NCHT, CHT), jnp.int32),
             pltpu.VMEM_SHARED((NAP, H), jnp.float32)]
            + [pltpu.VMEM((CHT, H), jnp.float32) for _ in range(KB)]
            + [pltpu.SemaphoreType.DMA for _ in range(KB)]
        ),
    )
    def k(a_hbm, p_hbm, src_hbm, dst_hbm, out_hbm, idx, tsp, *rest):
        bufs, sems = rest[:KB], rest[KB:]
        c = lax.axis_index("c")
        s = lax.axis_index("s")
        base = s * EPT

        @pl.when(c == 0)
        def _():
            pltpu.sync_copy(src_hbm.at[s], idx)
            pltpu.sync_copy(a_hbm.at[pl.ds(s * trows, trows)],
                            tsp.at[pl.ds(s * trows, trows)])

        @pl.when(c == 1)
        def _():
            pltpu.sync_copy(dst_hbm.at[s], idx)
            pltpu.sync_copy(p_hbm.at[pl.ds(s * prows, prows)],
                            tsp.at[pl.ds(s * prows, prows)])

        plsc.subcore_barrier()

        @pl.loop(0, NCHT // KB)
        def _(g):
            j0 = g * KB
            dG = [pltpu.async_copy(tsp.at[idx.at[j0 + b]], bufs[b], sems[b])
                  for b in range(KB)]
            dW = []
            for b in range(KB):
                dG[b].wait()
                dW.append(pltpu.async_copy(
                    bufs[b],
                    out_hbm.at[c, pl.ds(base + (j0 + b) * CHT, CHT)],
                    sems[b]))
            for b in range(KB):
                dW[b].wait()

    return k(tableA, tableP, srcT, dstT)


def _sc_scatter(messages, dst3, zeros_hbm):
    rows = PB // NS
    KS = 2

    @functools.partial(
        pl.kernel,
        out_type=jax.ShapeDtypeStruct((NC, PB, H), jnp.float32),
        mesh=_sc_mesh(),
        scratch_types=(
            [pltpu.VMEM((NCH, CH), jnp.int32),
             pltpu.VMEM_SHARED((PB, H), jnp.float32)]
            + [pltpu.VMEM((CH, H), jnp.float32) for _ in range(KS)]
            + [pltpu.SemaphoreType.DMA for _ in range(KS)]
        ),
    )
    def k(m_hbm, dst_hbm, z_hbm, out_hbm, didx, acc, *rest):
        bufs, sems = rest[:KS], rest[KS:]
        c = lax.axis_index("c")
        s = lax.axis_index("s")
        wid = s * NC + c
        base = wid * EPW
        pltpu.sync_copy(dst_hbm.at[wid], didx)
        pltpu.sync_copy(z_hbm.at[pl.ds(s * rows, rows)],
                        acc.at[pl.ds(s * rows, rows)])
        plsc.subcore_barrier()

        @pl.loop(0, NCH // KS)
        def _(g):
            j0 = g * KS
            dR = [pltpu.async_copy(
                      m_hbm.at[pl.ds(base + (j0 + b) * CH, CH)], bufs[b],
                      sems[b]) for b in range(KS)]
            dS = []
            for b in range(KS):
                dR[b].wait()
                dS.append(pltpu.async_copy(bufs[b], acc.at[didx.at[j0 + b]],
                                           sems[b], add=True))
            for b in range(KS):
                dS[b].wait()

        plsc.subcore_barrier()
        pltpu.sync_copy(acc.at[pl.ds(s * rows, rows)],
                        out_hbm.at[c, pl.ds(s * rows, rows)])

    return k(messages, dst3, zeros_hbm)


def kernel(atom_rep_0, atom_rep_1, atom_rep_2, probe_edges_features, params,
           probe_edges):
    f32 = jnp.float32
    src = probe_edges[:, 0].astype(jnp.int32)
    dst = probe_edges[:, 1].astype(jnp.int32)
    pad = EP - E
    src_p = jnp.concatenate([src, jnp.zeros((pad,), jnp.int32)])
    dst_p = jnp.concatenate([dst, jnp.full((pad,), NP, jnp.int32)])
    src3 = src_p.reshape(NW, NCH, CH)
    dst3 = dst_p.reshape(NW, NCH, CH)
    srcT = src_p.reshape(NS, NCHT, CHT)
    dstT = dst_p.reshape(NS, NCHT, CHT)
    d_pad = jnp.concatenate(
        [probe_edges_features.astype(f32),
         jnp.full((pad, 1), 2.0 * CUTOFF, f32)], axis=0)

    waT, wbT, b1s, m2T, b2s, f1T, bf1, f2T, bf2 = [], [], [], [], [], [], [], [], []
    for i in range(NI):
        p = params["msg"][i]
        w1 = p["m1"][0]
        waT.append(w1[:, :H].T)
        wbT.append(w1[:, H:].T)
        b1s.append(p["m1"][1][None, :])
        m2T.append(p["m2"][0].T)
        b2s.append(p["m2"][1][None, :])
        f1T.append(jnp.zeros((ESP, H), f32).at[:ES].set(p["f1"][0].T))
        bf1.append(p["f1"][1][None, :])
        f2T.append(p["f2"][0].T)
        bf2.append(p["f2"][1][None, :])
    gw = [[params["gate"][i][k][0].T for k in ("g1", "g2")] for i in range(NI)]
    gb = [[params["gate"][i][k][1][None, :] for k in ("g1", "g2")] for i in range(NI)]
    sw = [[params["state"][i][k][0].T for k in ("s1", "s2")] for i in range(NI)]
    sb = [[params["state"][i][k][1][None, :] for k in ("s1", "s2")] for i in range(NI)]
    r1T = params["readout"]["r1"][0].T
    br1 = params["readout"]["r1"][1][None, :]
    r2T = jnp.zeros((H, 8), f32).at[:, 0].set(params["readout"]["r2"][0][0])
    br2 = jnp.zeros((1, 8), f32).at[0, 0].set(params["readout"]["r2"][1][0])

    mu = jnp.arange(0.0, CUTOFF, STEP, dtype=f32)
    mu_pad = jnp.full((1, ESP), 1.0e3, f32).at[0, :ES].set(mu)

    zeros_pb = jnp.zeros((PB, H), f32)
    ones_row = jnp.ones((1, H), f32)

    atoms_s = jnp.zeros((NI, NAP, H), f32).at[:, :NA].set(
        jnp.stack([atom_rep_0, atom_rep_1, atom_rep_2]))
    A_all = _precompute_A(atoms_s, jnp.stack(waT))

    cc_col = _cc_precompute(d_pad.reshape(EP // H, H)).reshape(EP, 1)

    probe = zeros_pb
    P = None
    for i in range(NI):
        if i == 0:
            h = _sc_gather1(A_all[0], src3)
        else:
            h = _sc_gather2(A_all[i], P, srcT, dstT)
        messages = _edge_mlp(h, d_pad, cc_col, ones_row, b1s[i], m2T[i],
                             b2s[i], f1T[i], bf1[i], f2T[i], bf2[i], mu_pad)
        partials = _sc_scatter(messages, dst3, zeros_pb)
        gwi = [gw[i][0], gb[i][0], gw[i][1], gb[i][1]]
        swi = [sw[i][0], sb[i][0], sw[i][1], sb[i][1]]
        if i < NI - 1:
            probe, P = _probe_step(partials, probe, gwi, swi, wbT[i + 1])
        else:
            ro = _probe_final(partials, probe, gwi, swi, [r1T, br1, r2T, br2])
    return ro[:NP, 0][None, :]
